# baseline (speedup 1.0000x reference)
import functools

import jax
import jax.numpy as jnp
from jax import lax
from jax.experimental import pallas as pl
from jax.experimental.pallas import tpu as pltpu
from jax.experimental.pallas import tpu_sc as plsc

NC = 2
NS = 16
NW = NC * NS
CH = 128
BLK = 256
NUM_GRAPHS = 64
H = 8
B = 4
F32 = jnp.float32


def _sc_mesh():
  return plsc.VectorSubcoreMesh(core_axis_name="c", subcore_axis_name="s")


def _sc_degree(dst_r, n_pad):
  nchunk = dst_r.shape[1]
  rpt = n_pad // NS

  @functools.partial(
      pl.kernel,
      out_type=jax.ShapeDtypeStruct((NC, n_pad, 16), F32),
      mesh=_sc_mesh(),
      compiler_params=pltpu.CompilerParams(use_tc_tiling_on_sc=False),
      scratch_types=[
          pltpu.VMEM((nchunk, CH), jnp.int32),
          pltpu.VMEM((CH, 16), F32),
          pltpu.VMEM((16, 16), F32),
          pltpu.VMEM_SHARED((n_pad, 16), F32),
      ],
  )
  def deg_kernel(dst_hbm, out_hbm, dst_v, ones_v, zero_v, acc):
    cid = lax.axis_index("c")
    sid = lax.axis_index("s")
    wid = sid * NC + cid
    pltpu.sync_copy(dst_hbm.at[wid], dst_v)
    for i in range(16):
      zero_v[i, :] = jnp.zeros((16,), F32)

    def fill(i, carry):
      ones_v[i, :] = jnp.ones((16,), F32)
      return carry

    lax.fori_loop(0, CH, fill, 0)
    base = sid * rpt
    for k in range(rpt // 16):
      pltpu.sync_copy(zero_v, acc.at[pl.ds(base + k * 16, 16)])
    plsc.subcore_barrier()

    def step(j, carry):
      pltpu.sync_copy(ones_v, acc.at[dst_v.at[j]], add=True)
      return carry

    lax.fori_loop(0, nchunk, step, 0)
    plsc.subcore_barrier()
    pltpu.sync_copy(acc.at[pl.ds(base, rpt)], out_hbm.at[cid, pl.ds(base, rpt)])

  return deg_kernel(dst_r)


def _sc_edge_sum(y, src_r, dst_r, n_pad):
  nchunk = src_r.shape[1]
  rpt = n_pad // NS

  @functools.partial(
      pl.kernel,
      out_type=jax.ShapeDtypeStruct((NC, n_pad, 64), F32),
      mesh=_sc_mesh(),
      compiler_params=pltpu.CompilerParams(use_tc_tiling_on_sc=False),
      scratch_types=[
          pltpu.VMEM((nchunk, CH), jnp.int32),
          pltpu.VMEM((nchunk, CH), jnp.int32),
          pltpu.VMEM((CH, 64), F32),
          pltpu.VMEM((CH, 64), F32),
          pltpu.VMEM((16, 64), F32),
          pltpu.VMEM_SHARED((n_pad, 64), F32),
          pltpu.SemaphoreType.DMA,
          pltpu.SemaphoreType.DMA,
      ],
  )
  def scat_kernel(y_hbm, src_hbm, dst_hbm, out_hbm, src_v, dst_v, rows0,
                  rows1, zero_v, acc, sem0, sem1):
    cid = lax.axis_index("c")
    sid = lax.axis_index("s")
    wid = sid * NC + cid
    pltpu.sync_copy(src_hbm.at[wid], src_v)
    pltpu.sync_copy(dst_hbm.at[wid], dst_v)
    for i in range(16):
      for j in range(4):
        zero_v[i, pl.ds(j * 16, 16)] = jnp.zeros((16,), F32)
    base = sid * rpt
    for k in range(rpt // 16):
      pltpu.sync_copy(zero_v, acc.at[pl.ds(base + k * 16, 16)])
    plsc.subcore_barrier()

    def step(j, carry):
      pltpu.async_copy(y_hbm.at[src_v.at[j]], rows0, sem0).wait()
      pltpu.sync_copy(rows0, acc.at[dst_v.at[j]], add=True)
      return carry

    lax.fori_loop(0, nchunk, step, 0)
    plsc.subcore_barrier()
    pltpu.sync_copy(acc.at[pl.ds(base, rpt)], out_hbm.at[cid, pl.ds(base, rpt)])

  return scat_kernel(y, src_r, dst_r)


def _stage_a(h, wb, wc, bc, wr, br, deg0, deg1, dinv_in, n_pad, first):
  grid = n_pad // BLK
  hid = h.shape[1]

  def body(*refs):
    if first:
      (h_ref, wb_ref, wc_ref, bc_ref, wr_ref, br_ref, d0_ref, d1_ref,
       y_ref, comb_ref, res_ref, dinv_ref) = refs
      deg = d0_ref[:, 0:1] + d1_ref[:, 0:1] + 1.0
      dinv = lax.rsqrt(deg)
      dinv_ref[...] = jnp.broadcast_to(dinv, (BLK, 16))
    else:
      (h_ref, wb_ref, wc_ref, bc_ref, wr_ref, br_ref, dv_ref,
       y_ref, comb_ref, res_ref) = refs
      dinv = dv_ref[:, 0:1]
    hb = h_ref[...]
    y_ref[...] = jnp.dot(hb, wb_ref[...], preferred_element_type=F32) * dinv
    comb_ref[...] = jnp.dot(hb, wc_ref[...],
                            preferred_element_type=F32) + bc_ref[...]
    res_ref[...] = jnp.dot(hb, wr_ref[...],
                           preferred_element_type=F32) + br_ref[...]

  in_specs = [
      pl.BlockSpec((BLK, hid), lambda i: (i, 0)),
      pl.BlockSpec((hid, 64), lambda i: (0, 0)),
      pl.BlockSpec((hid, 32), lambda i: (0, 0)),
      pl.BlockSpec((1, 32), lambda i: (0, 0)),
      pl.BlockSpec((hid, 128), lambda i: (0, 0)),
      pl.BlockSpec((1, 128), lambda i: (0, 0)),
  ]
  out_shape = [
      jax.ShapeDtypeStruct((n_pad, 64), F32),
      jax.ShapeDtypeStruct((n_pad, 32), F32),
      jax.ShapeDtypeStruct((n_pad, 128), F32),
  ]
  out_specs = [
      pl.BlockSpec((BLK, 64), lambda i: (i, 0)),
      pl.BlockSpec((BLK, 32), lambda i: (i, 0)),
      pl.BlockSpec((BLK, 128), lambda i: (i, 0)),
  ]
  if first:
    in_specs += [pl.BlockSpec((BLK, 16), lambda i: (i, 0)),
                 pl.BlockSpec((BLK, 16), lambda i: (i, 0))]
    out_shape.append(jax.ShapeDtypeStruct((n_pad, 16), F32))
    out_specs.append(pl.BlockSpec((BLK, 16), lambda i: (i, 0)))
    args = (h, wb, wc, bc, wr, br, deg0, deg1)
  else:
    in_specs.append(pl.BlockSpec((BLK, 16), lambda i: (i, 0)))
    args = (h, wb, wc, bc, wr, br, dinv_in)

  return pl.pallas_call(
      body, grid=(grid,), in_specs=in_specs, out_specs=out_specs,
      out_shape=out_shape)(*args)


def _combine_body(y_ref, s0_ref, s1_ref, comb_ref, res_ref, dv_ref, bias_ref,
                  g_ref, be_ref):
  dinv = dv_ref[:, 0:1]
  agg = (y_ref[...] + s0_ref[...] + s1_ref[...]) * dinv
  comb = comb_ref[...]
  conv = jnp.zeros((BLK, 128), F32)
  ri_e = lax.broadcasted_iota(jnp.int32, (32, 128), 0)
  ci_e = lax.broadcasted_iota(jnp.int32, (32, 128), 1)
  ri_f = lax.broadcasted_iota(jnp.int32, (64, 128), 0)
  ci_f = lax.broadcasted_iota(jnp.int32, (64, 128), 1)
  for b in range(B):
    eb = jnp.where((ri_e % B == b) & (ci_e // 16 == ri_e // B), 1.0, 0.0)
    fb = jnp.where((ri_f // 16 == b) & (ci_f % 16 == ri_f % 16), 1.0, 0.0)
    ce = jnp.dot(comb, eb, preferred_element_type=F32)
    ae = jnp.dot(agg, fb, preferred_element_type=F32)
    conv = conv + ce * ae
  o = conv + bias_ref[...] + res_ref[...]
  mu = jnp.mean(o, axis=1, keepdims=True)
  var = jnp.mean((o - mu) ** 2, axis=1, keepdims=True)
  hn = (o - mu) / jnp.sqrt(var + 1e-5) * g_ref[...] + be_ref[...]
  return jnp.maximum(hn, 0.0)


def _stage_d(y, s0, s1, comb, res, dinv, bias, g, be, n_pad):
  grid = n_pad // BLK

  def body(y_ref, s0_ref, s1_ref, comb_ref, res_ref, dv_ref, bias_ref, g_ref,
           be_ref, out_ref):
    out_ref[...] = _combine_body(y_ref, s0_ref, s1_ref, comb_ref, res_ref,
                                 dv_ref, bias_ref, g_ref, be_ref)

  return pl.pallas_call(
      body, grid=(grid,),
      in_specs=[
          pl.BlockSpec((BLK, 64), lambda i: (i, 0)),
          pl.BlockSpec((BLK, 64), lambda i: (i, 0)),
          pl.BlockSpec((BLK, 64), lambda i: (i, 0)),
          pl.BlockSpec((BLK, 32), lambda i: (i, 0)),
          pl.BlockSpec((BLK, 128), lambda i: (i, 0)),
          pl.BlockSpec((BLK, 16), lambda i: (i, 0)),
          pl.BlockSpec((1, 128), lambda i: (0, 0)),
          pl.BlockSpec((1, 128), lambda i: (0, 0)),
          pl.BlockSpec((1, 128), lambda i: (0, 0)),
      ],
      out_specs=pl.BlockSpec((BLK, 128), lambda i: (i, 0)),
      out_shape=jax.ShapeDtypeStruct((n_pad, 128), F32),
  )(y, s0, s1, comb, res, dinv, bias, g, be)


def _stage_d_pool(y, s0, s1, comb, res, dinv, bias, g, be, batch3, wf, n_pad):
  grid = n_pad // BLK

  def body(y_ref, s0_ref, s1_ref, comb_ref, res_ref, dv_ref, bias_ref, g_ref,
           be_ref, batch_ref, wf_ref, out_ref, sums, cnts):
    i = pl.program_id(0)

    @pl.when(i == 0)
    def _init():
      sums[...] = jnp.zeros((NUM_GRAPHS, 128), F32)
      cnts[...] = jnp.zeros((NUM_GRAPHS, 128), F32)

    h2 = _combine_body(y_ref, s0_ref, s1_ref, comb_ref, res_ref, dv_ref,
                       bias_ref, g_ref, be_ref)
    brow = batch_ref[0]
    gi = lax.broadcasted_iota(jnp.int32, (NUM_GRAPHS, BLK), 0)
    oh = jnp.where(gi == brow, 1.0, 0.0)
    sums[...] += jnp.dot(oh, h2, preferred_element_type=F32)
    cnts[...] += jnp.broadcast_to(
        jnp.sum(oh, axis=1, keepdims=True), (NUM_GRAPHS, 128))

    @pl.when(i == grid - 1)
    def _fin():
      pooled = sums[...] / jnp.maximum(cnts[...], 1.0)
      out_ref[...] = jnp.dot(pooled, wf_ref[...], preferred_element_type=F32)

  return pl.pallas_call(
      body, grid=(grid,),
      in_specs=[
          pl.BlockSpec((BLK, 64), lambda i: (i, 0)),
          pl.BlockSpec((BLK, 64), lambda i: (i, 0)),
          pl.BlockSpec((BLK, 64), lambda i: (i, 0)),
          pl.BlockSpec((BLK, 32), lambda i: (i, 0)),
          pl.BlockSpec((BLK, 128), lambda i: (i, 0)),
          pl.BlockSpec((BLK, 16), lambda i: (i, 0)),
          pl.BlockSpec((1, 128), lambda i: (0, 0)),
          pl.BlockSpec((1, 128), lambda i: (0, 0)),
          pl.BlockSpec((1, 128), lambda i: (0, 0)),
          pl.BlockSpec((1, 1, BLK), lambda i: (i, 0, 0)),
          pl.BlockSpec((128, 8), lambda i: (0, 0)),
      ],
      out_specs=pl.BlockSpec((NUM_GRAPHS, 8), lambda i: (0, 0)),
      out_shape=jax.ShapeDtypeStruct((NUM_GRAPHS, 8), F32),
      compiler_params=pltpu.CompilerParams(
          dimension_semantics=("arbitrary",)),
      scratch_shapes=[pltpu.VMEM((NUM_GRAPHS, 128), F32),
                      pltpu.VMEM((NUM_GRAPHS, 128), F32)],
  )(y, s0, s1, comb, res, dinv, bias, g, be, batch3, wf)


def kernel(x, edge_index, batch, Wb0, Wc0, bc0, bias0, Wr0, br0, g0, be0,
           Wb1, Wc1, bc1, bias1, Wr1, br1, g1, be1, Wf):
  n = x.shape[0]
  e = edge_index.shape[1]
  n_pad = ((n + BLK - 1) // BLK) * BLK
  nchunk = -(-e // (NW * CH))
  nchunk += nchunk % 2
  e_pad = NW * nchunk * CH

  xp = jnp.pad(x, ((0, n_pad - n), (0, 0)))
  batch_p = jnp.concatenate(
      [batch, jnp.full((n_pad - n,), NUM_GRAPHS, jnp.int32)])
  batch3 = batch_p.reshape(n_pad // BLK, 1, BLK)
  pad_e = jnp.full((e_pad - e,), n_pad - 1, jnp.int32)
  src_r = jnp.concatenate([edge_index[0], pad_e]).reshape(NW, nchunk, CH)
  dst_r = jnp.concatenate([edge_index[1], pad_e]).reshape(NW, nchunk, CH)

  bc0r, bc1r = bc0.reshape(1, -1), bc1.reshape(1, -1)
  row = lambda v: v.reshape(1, -1)

  deg_p = _sc_degree(dst_r, n_pad)

  y0, comb0, res0, dinv = _stage_a(
      xp, Wb0, Wc0, bc0r, Wr0, row(br0), deg_p[0], deg_p[1], None, n_pad,
      first=True)
  s_p0 = _sc_edge_sum(y0, src_r, dst_r, n_pad)
  h1 = _stage_d(y0, s_p0[0], s_p0[1], comb0, res0, dinv, row(bias0), row(g0),
                row(be0), n_pad)

  y1, comb1, res1 = _stage_a(
      h1, Wb1, Wc1, bc1r, Wr1, row(br1), None, None, dinv, n_pad, first=False)
  s_p1 = _sc_edge_sum(y1, src_r, dst_r, n_pad)
  out = _stage_d_pool(y1, s_p1[0], s_p1[1], comb1, res1, dinv, row(bias1),
                      row(g1), row(be1), batch3, Wf, n_pad)
  return out


# speedup vs baseline: 1.6564x; 1.6564x over previous
import functools

import jax
import jax.numpy as jnp
from jax import lax
from jax.experimental import pallas as pl
from jax.experimental.pallas import tpu as pltpu
from jax.experimental.pallas import tpu_sc as plsc

NC = 2
NS = 16
NW = NC * NS
CH = 128
BLK = 256
NUM_GRAPHS = 64
H = 8
B = 4
F32 = jnp.float32


def _sc_mesh():
  return plsc.VectorSubcoreMesh(core_axis_name="c", subcore_axis_name="s")


def _sc_degree(dst_r, n_pad):
  nchunk = dst_r.shape[1]
  rpt = n_pad // NS

  @functools.partial(
      pl.kernel,
      out_type=jax.ShapeDtypeStruct((NC, n_pad, 16), F32),
      mesh=_sc_mesh(),
      compiler_params=pltpu.CompilerParams(use_tc_tiling_on_sc=False),
      scratch_types=[
          pltpu.VMEM((nchunk, CH), jnp.int32),
          pltpu.VMEM((CH, 16), F32),
          pltpu.VMEM((16, 16), F32),
          pltpu.VMEM_SHARED((n_pad, 16), F32),
      ],
  )
  def deg_kernel(dst_hbm, out_hbm, dst_v, ones_v, zero_v, acc):
    cid = lax.axis_index("c")
    sid = lax.axis_index("s")
    wid = sid * NC + cid
    pltpu.sync_copy(dst_hbm.at[wid], dst_v)
    for i in range(16):
      zero_v[i, :] = jnp.zeros((16,), F32)

    def fill(i, carry):
      ones_v[i, :] = jnp.ones((16,), F32)
      return carry

    lax.fori_loop(0, CH, fill, 0)
    base = sid * rpt
    for k in range(rpt // 16):
      pltpu.sync_copy(zero_v, acc.at[pl.ds(base + k * 16, 16)])
    plsc.subcore_barrier()

    def step(j, carry):
      pltpu.sync_copy(ones_v, acc.at[dst_v.at[j]], add=True)
      return carry

    lax.fori_loop(0, nchunk, step, 0)
    plsc.subcore_barrier()
    pltpu.sync_copy(acc.at[pl.ds(base, rpt)], out_hbm.at[cid, pl.ds(base, rpt)])

  return deg_kernel(dst_r)


def _sc_edge_sum(y, src_r, dst_r, n_pad):
  nchunk = src_r.shape[1]
  rpt = n_pad // NS

  @functools.partial(
      pl.kernel,
      out_type=jax.ShapeDtypeStruct((NC, n_pad, 64), F32),
      mesh=_sc_mesh(),
      compiler_params=pltpu.CompilerParams(use_tc_tiling_on_sc=False),
      scratch_types=[
          pltpu.VMEM((nchunk, CH), jnp.int32),
          pltpu.VMEM((nchunk, CH), jnp.int32),
          pltpu.VMEM((CH, 64), F32),
          pltpu.VMEM((CH, 64), F32),
          pltpu.VMEM((16, 64), F32),
          pltpu.VMEM_SHARED((n_pad, 64), F32),
          pltpu.SemaphoreType.DMA,
          pltpu.SemaphoreType.DMA,
      ],
  )
  def scat_kernel(y_hbm, src_hbm, dst_hbm, out_hbm, src_v, dst_v, rows0,
                  rows1, zero_v, acc, sem0, sem1):
    cid = lax.axis_index("c")
    sid = lax.axis_index("s")
    wid = sid * NC + cid
    pltpu.sync_copy(src_hbm.at[wid], src_v)
    pltpu.sync_copy(dst_hbm.at[wid], dst_v)
    for i in range(16):
      for j in range(4):
        zero_v[i, pl.ds(j * 16, 16)] = jnp.zeros((16,), F32)
    base = sid * rpt
    for k in range(rpt // 16):
      pltpu.sync_copy(zero_v, acc.at[pl.ds(base + k * 16, 16)])
    plsc.subcore_barrier()

    def step(j, carry):
      pltpu.async_copy(y_hbm.at[src_v.at[j]], rows0, sem0).wait()
      pltpu.sync_copy(rows0, acc.at[dst_v.at[j]], add=True)
      return carry

    lax.fori_loop(0, nchunk, step, 0)
    plsc.subcore_barrier()
    pltpu.sync_copy(acc.at[pl.ds(base, rpt)], out_hbm.at[cid, pl.ds(base, rpt)])

  return scat_kernel(y, src_r, dst_r)


def _stage_a(h, wb, wc, bc, wr, br, deg0, deg1, dinv_in, n_pad, first):
  grid = n_pad // BLK
  hid = h.shape[1]

  def body(*refs):
    if first:
      (h_ref, wb_ref, wc_ref, bc_ref, wr_ref, br_ref, d0_ref, d1_ref,
       y_ref, comb_ref, res_ref, dinv_ref) = refs
      deg = d0_ref[:, 0:1] + d1_ref[:, 0:1] + 1.0
      dinv = lax.rsqrt(deg)
      dinv_ref[...] = jnp.broadcast_to(dinv, (BLK, 16))
    else:
      (h_ref, wb_ref, wc_ref, bc_ref, wr_ref, br_ref, dv_ref,
       y_ref, comb_ref, res_ref) = refs
      dinv = dv_ref[:, 0:1]
    hb = h_ref[...]
    y_ref[...] = jnp.dot(hb, wb_ref[...], preferred_element_type=F32) * dinv
    comb_ref[...] = jnp.dot(hb, wc_ref[...],
                            preferred_element_type=F32) + bc_ref[...]
    res_ref[...] = jnp.dot(hb, wr_ref[...],
                           preferred_element_type=F32) + br_ref[...]

  in_specs = [
      pl.BlockSpec((BLK, hid), lambda i: (i, 0)),
      pl.BlockSpec((hid, 64), lambda i: (0, 0)),
      pl.BlockSpec((hid, 32), lambda i: (0, 0)),
      pl.BlockSpec((1, 32), lambda i: (0, 0)),
      pl.BlockSpec((hid, 128), lambda i: (0, 0)),
      pl.BlockSpec((1, 128), lambda i: (0, 0)),
  ]
  out_shape = [
      jax.ShapeDtypeStruct((n_pad, 64), F32),
      jax.ShapeDtypeStruct((n_pad, 32), F32),
      jax.ShapeDtypeStruct((n_pad, 128), F32),
  ]
  out_specs = [
      pl.BlockSpec((BLK, 64), lambda i: (i, 0)),
      pl.BlockSpec((BLK, 32), lambda i: (i, 0)),
      pl.BlockSpec((BLK, 128), lambda i: (i, 0)),
  ]
  if first:
    in_specs += [pl.BlockSpec((BLK, 16), lambda i: (i, 0)),
                 pl.BlockSpec((BLK, 16), lambda i: (i, 0))]
    out_shape.append(jax.ShapeDtypeStruct((n_pad, 16), F32))
    out_specs.append(pl.BlockSpec((BLK, 16), lambda i: (i, 0)))
    args = (h, wb, wc, bc, wr, br, deg0, deg1)
  else:
    in_specs.append(pl.BlockSpec((BLK, 16), lambda i: (i, 0)))
    args = (h, wb, wc, bc, wr, br, dinv_in)

  return pl.pallas_call(
      body, grid=(grid,), in_specs=in_specs, out_specs=out_specs,
      out_shape=out_shape)(*args)


def _combine_body(y_ref, s0_ref, s1_ref, comb_ref, res_ref, dv_ref, bias_ref,
                  g_ref, be_ref):
  dinv = dv_ref[:, 0:1]
  agg = (y_ref[...] + s0_ref[...] + s1_ref[...]) * dinv
  comb = comb_ref[...]
  conv = jnp.zeros((BLK, 128), F32)
  ri_e = lax.broadcasted_iota(jnp.int32, (32, 128), 0)
  ci_e = lax.broadcasted_iota(jnp.int32, (32, 128), 1)
  ri_f = lax.broadcasted_iota(jnp.int32, (64, 128), 0)
  ci_f = lax.broadcasted_iota(jnp.int32, (64, 128), 1)
  for b in range(B):
    eb = jnp.where((ri_e % B == b) & (ci_e // 16 == ri_e // B), 1.0, 0.0)
    fb = jnp.where((ri_f // 16 == b) & (ci_f % 16 == ri_f % 16), 1.0, 0.0)
    ce = jnp.dot(comb, eb, preferred_element_type=F32)
    ae = jnp.dot(agg, fb, preferred_element_type=F32)
    conv = conv + ce * ae
  o = conv + bias_ref[...] + res_ref[...]
  mu = jnp.mean(o, axis=1, keepdims=True)
  var = jnp.mean((o - mu) ** 2, axis=1, keepdims=True)
  hn = (o - mu) / jnp.sqrt(var + 1e-5) * g_ref[...] + be_ref[...]
  return jnp.maximum(hn, 0.0)


def _stage_d(y, s0, s1, comb, res, dinv, bias, g, be, n_pad):
  grid = n_pad // BLK

  def body(y_ref, s0_ref, s1_ref, comb_ref, res_ref, dv_ref, bias_ref, g_ref,
           be_ref, out_ref):
    out_ref[...] = _combine_body(y_ref, s0_ref, s1_ref, comb_ref, res_ref,
                                 dv_ref, bias_ref, g_ref, be_ref)

  return pl.pallas_call(
      body, grid=(grid,),
      in_specs=[
          pl.BlockSpec((BLK, 64), lambda i: (i, 0)),
          pl.BlockSpec((BLK, 64), lambda i: (i, 0)),
          pl.BlockSpec((BLK, 64), lambda i: (i, 0)),
          pl.BlockSpec((BLK, 32), lambda i: (i, 0)),
          pl.BlockSpec((BLK, 128), lambda i: (i, 0)),
          pl.BlockSpec((BLK, 16), lambda i: (i, 0)),
          pl.BlockSpec((1, 128), lambda i: (0, 0)),
          pl.BlockSpec((1, 128), lambda i: (0, 0)),
          pl.BlockSpec((1, 128), lambda i: (0, 0)),
      ],
      out_specs=pl.BlockSpec((BLK, 128), lambda i: (i, 0)),
      out_shape=jax.ShapeDtypeStruct((n_pad, 128), F32),
  )(y, s0, s1, comb, res, dinv, bias, g, be)


def _stage_d_pool(y, s0, s1, comb, res, dinv, bias, g, be, batch3, wf, n_pad):
  grid = n_pad // BLK

  def body(y_ref, s0_ref, s1_ref, comb_ref, res_ref, dv_ref, bias_ref, g_ref,
           be_ref, batch_ref, wf_ref, out_ref, sums, cnts):
    i = pl.program_id(0)

    @pl.when(i == 0)
    def _init():
      sums[...] = jnp.zeros((NUM_GRAPHS, 128), F32)
      cnts[...] = jnp.zeros((NUM_GRAPHS, 128), F32)

    h2 = _combine_body(y_ref, s0_ref, s1_ref, comb_ref, res_ref, dv_ref,
                       bias_ref, g_ref, be_ref)
    brow = batch_ref[0]
    gi = lax.broadcasted_iota(jnp.int32, (NUM_GRAPHS, BLK), 0)
    oh = jnp.where(gi == brow, 1.0, 0.0)
    sums[...] += jnp.dot(oh, h2, preferred_element_type=F32)
    cnts[...] += jnp.broadcast_to(
        jnp.sum(oh, axis=1, keepdims=True), (NUM_GRAPHS, 128))

    @pl.when(i == grid - 1)
    def _fin():
      pooled = sums[...] / jnp.maximum(cnts[...], 1.0)
      out_ref[...] = jnp.dot(pooled, wf_ref[...], preferred_element_type=F32)

  return pl.pallas_call(
      body, grid=(grid,),
      in_specs=[
          pl.BlockSpec((BLK, 64), lambda i: (i, 0)),
          pl.BlockSpec((BLK, 64), lambda i: (i, 0)),
          pl.BlockSpec((BLK, 64), lambda i: (i, 0)),
          pl.BlockSpec((BLK, 32), lambda i: (i, 0)),
          pl.BlockSpec((BLK, 128), lambda i: (i, 0)),
          pl.BlockSpec((BLK, 16), lambda i: (i, 0)),
          pl.BlockSpec((1, 128), lambda i: (0, 0)),
          pl.BlockSpec((1, 128), lambda i: (0, 0)),
          pl.BlockSpec((1, 128), lambda i: (0, 0)),
          pl.BlockSpec((1, 1, BLK), lambda i: (i, 0, 0)),
          pl.BlockSpec((128, 8), lambda i: (0, 0)),
      ],
      out_specs=pl.BlockSpec((NUM_GRAPHS, 8), lambda i: (0, 0)),
      out_shape=jax.ShapeDtypeStruct((NUM_GRAPHS, 8), F32),
      compiler_params=pltpu.CompilerParams(
          dimension_semantics=("arbitrary",)),
      scratch_shapes=[pltpu.VMEM((NUM_GRAPHS, 128), F32),
                      pltpu.VMEM((NUM_GRAPHS, 128), F32)],
  )(y, s0, s1, comb, res, dinv, bias, g, be, batch3, wf)


def kernel(x, edge_index, batch, Wb0, Wc0, bc0, bias0, Wr0, br0, g0, be0,
           Wb1, Wc1, bc1, bias1, Wr1, br1, g1, be1, Wf):
  n = x.shape[0]
  e = edge_index.shape[1]
  n_pad = ((n + BLK - 1) // BLK) * BLK
  nchunk = -(-e // (NW * CH))
  e_pad = NW * nchunk * CH

  xp = jnp.pad(x, ((0, n_pad - n), (0, 0)))
  batch_p = jnp.concatenate(
      [batch, jnp.full((n_pad - n,), NUM_GRAPHS, jnp.int32)])
  batch3 = batch_p.reshape(n_pad // BLK, 1, BLK)
  pad_e = n + jnp.arange(e_pad - e, dtype=jnp.int32) % (n_pad - n)
  src_r = jnp.concatenate([edge_index[0], pad_e]).reshape(NW, nchunk, CH)
  dst_r = jnp.concatenate([edge_index[1], pad_e]).reshape(NW, nchunk, CH)

  bc0r, bc1r = bc0.reshape(1, -1), bc1.reshape(1, -1)
  row = lambda v: v.reshape(1, -1)

  deg_p = _sc_degree(dst_r, n_pad)

  y0, comb0, res0, dinv = _stage_a(
      xp, Wb0, Wc0, bc0r, Wr0, row(br0), deg_p[0], deg_p[1], None, n_pad,
      first=True)
  s_p0 = _sc_edge_sum(y0, src_r, dst_r, n_pad)
  h1 = _stage_d(y0, s_p0[0], s_p0[1], comb0, res0, dinv, row(bias0), row(g0),
                row(be0), n_pad)

  y1, comb1, res1 = _stage_a(
      h1, Wb1, Wc1, bc1r, Wr1, row(br1), None, None, dinv, n_pad, first=False)
  s_p1 = _sc_edge_sum(y1, src_r, dst_r, n_pad)
  out = _stage_d_pool(y1, s_p1[0], s_p1[1], comb1, res1, dinv, row(bias1),
                      row(g1), row(be1), batch3, Wf, n_pad)
  return out


# speedup vs baseline: 2.0770x; 1.2539x over previous
import functools

import jax
import jax.numpy as jnp
from jax import lax
from jax.experimental import pallas as pl
from jax.experimental.pallas import tpu as pltpu
from jax.experimental.pallas import tpu_sc as plsc

NC = 2
NS = 16
NW = NC * NS
CH = 128
BLK = 256
NUM_GRAPHS = 64
H = 8
B = 4
F32 = jnp.float32


def _sc_mesh():
  return plsc.VectorSubcoreMesh(core_axis_name="c", subcore_axis_name="s")


def _sc_degree(dst_r, n_pad):
  nchunk = dst_r.shape[1]
  rpt = n_pad // NS

  @functools.partial(
      pl.kernel,
      out_type=jax.ShapeDtypeStruct((NC, n_pad, 16), F32),
      mesh=_sc_mesh(),
      compiler_params=pltpu.CompilerParams(use_tc_tiling_on_sc=False),
      scratch_types=[
          pltpu.VMEM((nchunk, CH), jnp.int32),
          pltpu.VMEM((CH, 16), F32),
          pltpu.VMEM((16, 16), F32),
          pltpu.VMEM_SHARED((n_pad, 16), F32),
      ],
  )
  def deg_kernel(dst_hbm, out_hbm, dst_v, ones_v, zero_v, acc):
    cid = lax.axis_index("c")
    sid = lax.axis_index("s")
    wid = sid * NC + cid
    pltpu.sync_copy(dst_hbm.at[wid], dst_v)
    for i in range(16):
      zero_v[i, :] = jnp.zeros((16,), F32)

    def fill(i, carry):
      ones_v[i, :] = jnp.ones((16,), F32)
      return carry

    lax.fori_loop(0, CH, fill, 0)
    base = sid * rpt
    for k in range(rpt // 16):
      pltpu.sync_copy(zero_v, acc.at[pl.ds(base + k * 16, 16)])
    plsc.subcore_barrier()

    def step(j, carry):
      pltpu.sync_copy(ones_v, acc.at[dst_v.at[j]], add=True)
      return carry

    lax.fori_loop(0, nchunk, step, 0)
    plsc.subcore_barrier()
    pltpu.sync_copy(acc.at[pl.ds(base, rpt)], out_hbm.at[cid, pl.ds(base, rpt)])

  return deg_kernel(dst_r)


def _sc_edge_sum(y, src_r, dst_r, n_pad):
  nchunk = src_r.shape[1]
  rpt = n_pad // NS

  @functools.partial(
      pl.kernel,
      out_type=jax.ShapeDtypeStruct((NC, n_pad, 64), F32),
      mesh=_sc_mesh(),
      compiler_params=pltpu.CompilerParams(use_tc_tiling_on_sc=False),
      scratch_types=[
          pltpu.VMEM((nchunk, CH), jnp.int32),
          pltpu.VMEM((nchunk, CH), jnp.int32),
          pltpu.VMEM((CH, 64), F32),
          pltpu.VMEM((CH, 64), F32),
          pltpu.VMEM((16, 64), F32),
          pltpu.VMEM_SHARED((n_pad, 64), F32),
          pltpu.SemaphoreType.DMA,
          pltpu.SemaphoreType.DMA,
      ],
  )
  def scat_kernel(y_hbm, src_hbm, dst_hbm, out_hbm, src_v, dst_v, rows0,
                  rows1, zero_v, acc, sem0, sem1):
    cid = lax.axis_index("c")
    sid = lax.axis_index("s")
    wid = sid * NC + cid
    pltpu.sync_copy(src_hbm.at[wid], src_v)
    pltpu.sync_copy(dst_hbm.at[wid], dst_v)
    for i in range(16):
      for j in range(4):
        zero_v[i, pl.ds(j * 16, 16)] = jnp.zeros((16,), F32)
    base = sid * rpt
    for k in range(rpt // 16):
      pltpu.sync_copy(zero_v, acc.at[pl.ds(base + k * 16, 16)])
    plsc.subcore_barrier()

    pltpu.async_copy(y_hbm.at[src_v.at[0]], rows0, sem0)

    def pair(i, carry):
      j = 2 * i
      pltpu.async_copy(y_hbm.at[src_v.at[j + 1]], rows1, sem1)
      pltpu.make_async_copy(y_hbm.at[pl.ds(0, CH)], rows0, sem0).wait()
      pltpu.sync_copy(rows0, acc.at[dst_v.at[j]], add=True)
      pltpu.async_copy(y_hbm.at[src_v.at[j + 2]], rows0, sem0)
      pltpu.make_async_copy(y_hbm.at[pl.ds(0, CH)], rows1, sem1).wait()
      pltpu.sync_copy(rows1, acc.at[dst_v.at[j + 1]], add=True)
      return carry

    lax.fori_loop(0, (nchunk - 1) // 2, pair, 0)
    pltpu.make_async_copy(y_hbm.at[pl.ds(0, CH)], rows0, sem0).wait()
    pltpu.sync_copy(rows0, acc.at[dst_v.at[nchunk - 1]], add=True)
    plsc.subcore_barrier()
    pltpu.sync_copy(acc.at[pl.ds(base, rpt)], out_hbm.at[cid, pl.ds(base, rpt)])

  return scat_kernel(y, src_r, dst_r)


def _stage_a(h, wb, wc, bc, wr, br, deg0, deg1, dinv_in, n_pad, first):
  grid = n_pad // BLK
  hid = h.shape[1]

  def body(*refs):
    if first:
      (h_ref, wb_ref, wc_ref, bc_ref, wr_ref, br_ref, d0_ref, d1_ref,
       y_ref, comb_ref, res_ref, dinv_ref) = refs
      deg = d0_ref[:, 0:1] + d1_ref[:, 0:1] + 1.0
      dinv = lax.rsqrt(deg)
      dinv_ref[...] = jnp.broadcast_to(dinv, (BLK, 16))
    else:
      (h_ref, wb_ref, wc_ref, bc_ref, wr_ref, br_ref, dv_ref,
       y_ref, comb_ref, res_ref) = refs
      dinv = dv_ref[:, 0:1]
    hb = h_ref[...]
    y_ref[...] = jnp.dot(hb, wb_ref[...], preferred_element_type=F32) * dinv
    comb_ref[...] = jnp.dot(hb, wc_ref[...],
                            preferred_element_type=F32) + bc_ref[...]
    res_ref[...] = jnp.dot(hb, wr_ref[...],
                           preferred_element_type=F32) + br_ref[...]

  in_specs = [
      pl.BlockSpec((BLK, hid), lambda i: (i, 0)),
      pl.BlockSpec((hid, 64), lambda i: (0, 0)),
      pl.BlockSpec((hid, 32), lambda i: (0, 0)),
      pl.BlockSpec((1, 32), lambda i: (0, 0)),
      pl.BlockSpec((hid, 128), lambda i: (0, 0)),
      pl.BlockSpec((1, 128), lambda i: (0, 0)),
  ]
  out_shape = [
      jax.ShapeDtypeStruct((n_pad, 64), F32),
      jax.ShapeDtypeStruct((n_pad, 32), F32),
      jax.ShapeDtypeStruct((n_pad, 128), F32),
  ]
  out_specs = [
      pl.BlockSpec((BLK, 64), lambda i: (i, 0)),
      pl.BlockSpec((BLK, 32), lambda i: (i, 0)),
      pl.BlockSpec((BLK, 128), lambda i: (i, 0)),
  ]
  if first:
    in_specs += [pl.BlockSpec((BLK, 16), lambda i: (i, 0)),
                 pl.BlockSpec((BLK, 16), lambda i: (i, 0))]
    out_shape.append(jax.ShapeDtypeStruct((n_pad, 16), F32))
    out_specs.append(pl.BlockSpec((BLK, 16), lambda i: (i, 0)))
    args = (h, wb, wc, bc, wr, br, deg0, deg1)
  else:
    in_specs.append(pl.BlockSpec((BLK, 16), lambda i: (i, 0)))
    args = (h, wb, wc, bc, wr, br, dinv_in)

  return pl.pallas_call(
      body, grid=(grid,), in_specs=in_specs, out_specs=out_specs,
      out_shape=out_shape)(*args)


def _combine_body(y_ref, s0_ref, s1_ref, comb_ref, res_ref, dv_ref, bias_ref,
                  g_ref, be_ref):
  dinv = dv_ref[:, 0:1]
  agg = (y_ref[...] + s0_ref[...] + s1_ref[...]) * dinv
  comb = comb_ref[...]
  conv = jnp.zeros((BLK, 128), F32)
  ri_e = lax.broadcasted_iota(jnp.int32, (32, 128), 0)
  ci_e = lax.broadcasted_iota(jnp.int32, (32, 128), 1)
  ri_f = lax.broadcasted_iota(jnp.int32, (64, 128), 0)
  ci_f = lax.broadcasted_iota(jnp.int32, (64, 128), 1)
  for b in range(B):
    eb = jnp.where((ri_e % B == b) & (ci_e // 16 == ri_e // B), 1.0, 0.0)
    fb = jnp.where((ri_f // 16 == b) & (ci_f % 16 == ri_f % 16), 1.0, 0.0)
    ce = jnp.dot(comb, eb, preferred_element_type=F32)
    ae = jnp.dot(agg, fb, preferred_element_type=F32)
    conv = conv + ce * ae
  o = conv + bias_ref[...] + res_ref[...]
  mu = jnp.mean(o, axis=1, keepdims=True)
  var = jnp.mean((o - mu) ** 2, axis=1, keepdims=True)
  hn = (o - mu) / jnp.sqrt(var + 1e-5) * g_ref[...] + be_ref[...]
  return jnp.maximum(hn, 0.0)


def _stage_d(y, s0, s1, comb, res, dinv, bias, g, be, n_pad):
  grid = n_pad // BLK

  def body(y_ref, s0_ref, s1_ref, comb_ref, res_ref, dv_ref, bias_ref, g_ref,
           be_ref, out_ref):
    out_ref[...] = _combine_body(y_ref, s0_ref, s1_ref, comb_ref, res_ref,
                                 dv_ref, bias_ref, g_ref, be_ref)

  return pl.pallas_call(
      body, grid=(grid,),
      in_specs=[
          pl.BlockSpec((BLK, 64), lambda i: (i, 0)),
          pl.BlockSpec((BLK, 64), lambda i: (i, 0)),
          pl.BlockSpec((BLK, 64), lambda i: (i, 0)),
          pl.BlockSpec((BLK, 32), lambda i: (i, 0)),
          pl.BlockSpec((BLK, 128), lambda i: (i, 0)),
          pl.BlockSpec((BLK, 16), lambda i: (i, 0)),
          pl.BlockSpec((1, 128), lambda i: (0, 0)),
          pl.BlockSpec((1, 128), lambda i: (0, 0)),
          pl.BlockSpec((1, 128), lambda i: (0, 0)),
      ],
      out_specs=pl.BlockSpec((BLK, 128), lambda i: (i, 0)),
      out_shape=jax.ShapeDtypeStruct((n_pad, 128), F32),
  )(y, s0, s1, comb, res, dinv, bias, g, be)


def _stage_d_pool(y, s0, s1, comb, res, dinv, bias, g, be, batch3, wf, n_pad):
  grid = n_pad // BLK

  def body(y_ref, s0_ref, s1_ref, comb_ref, res_ref, dv_ref, bias_ref, g_ref,
           be_ref, batch_ref, wf_ref, out_ref, sums, cnts):
    i = pl.program_id(0)

    @pl.when(i == 0)
    def _init():
      sums[...] = jnp.zeros((NUM_GRAPHS, 128), F32)
      cnts[...] = jnp.zeros((NUM_GRAPHS, 128), F32)

    h2 = _combine_body(y_ref, s0_ref, s1_ref, comb_ref, res_ref, dv_ref,
                       bias_ref, g_ref, be_ref)
    brow = batch_ref[0]
    gi = lax.broadcasted_iota(jnp.int32, (NUM_GRAPHS, BLK), 0)
    oh = jnp.where(gi == brow, 1.0, 0.0)
    sums[...] += jnp.dot(oh, h2, preferred_element_type=F32)
    cnts[...] += jnp.broadcast_to(
        jnp.sum(oh, axis=1, keepdims=True), (NUM_GRAPHS, 128))

    @pl.when(i == grid - 1)
    def _fin():
      pooled = sums[...] / jnp.maximum(cnts[...], 1.0)
      out_ref[...] = jnp.dot(pooled, wf_ref[...], preferred_element_type=F32)

  return pl.pallas_call(
      body, grid=(grid,),
      in_specs=[
          pl.BlockSpec((BLK, 64), lambda i: (i, 0)),
          pl.BlockSpec((BLK, 64), lambda i: (i, 0)),
          pl.BlockSpec((BLK, 64), lambda i: (i, 0)),
          pl.BlockSpec((BLK, 32), lambda i: (i, 0)),
          pl.BlockSpec((BLK, 128), lambda i: (i, 0)),
          pl.BlockSpec((BLK, 16), lambda i: (i, 0)),
          pl.BlockSpec((1, 128), lambda i: (0, 0)),
          pl.BlockSpec((1, 128), lambda i: (0, 0)),
          pl.BlockSpec((1, 128), lambda i: (0, 0)),
          pl.BlockSpec((1, 1, BLK), lambda i: (i, 0, 0)),
          pl.BlockSpec((128, 8), lambda i: (0, 0)),
      ],
      out_specs=pl.BlockSpec((NUM_GRAPHS, 8), lambda i: (0, 0)),
      out_shape=jax.ShapeDtypeStruct((NUM_GRAPHS, 8), F32),
      compiler_params=pltpu.CompilerParams(
          dimension_semantics=("arbitrary",)),
      scratch_shapes=[pltpu.VMEM((NUM_GRAPHS, 128), F32),
                      pltpu.VMEM((NUM_GRAPHS, 128), F32)],
  )(y, s0, s1, comb, res, dinv, bias, g, be, batch3, wf)


def kernel(x, edge_index, batch, Wb0, Wc0, bc0, bias0, Wr0, br0, g0, be0,
           Wb1, Wc1, bc1, bias1, Wr1, br1, g1, be1, Wf):
  n = x.shape[0]
  e = edge_index.shape[1]
  n_pad = ((n + BLK - 1) // BLK) * BLK
  nchunk = -(-e // (NW * CH))
  nchunk += 1 - nchunk % 2
  e_pad = NW * nchunk * CH

  xp = jnp.pad(x, ((0, n_pad - n), (0, 0)))
  batch_p = jnp.concatenate(
      [batch, jnp.full((n_pad - n,), NUM_GRAPHS, jnp.int32)])
  batch3 = batch_p.reshape(n_pad // BLK, 1, BLK)
  pad_e = n + jnp.arange(e_pad - e, dtype=jnp.int32) % (n_pad - n)
  src_r = jnp.concatenate([edge_index[0], pad_e]).reshape(NW, nchunk, CH)
  dst_r = jnp.concatenate([edge_index[1], pad_e]).reshape(NW, nchunk, CH)

  bc0r, bc1r = bc0.reshape(1, -1), bc1.reshape(1, -1)
  row = lambda v: v.reshape(1, -1)

  deg_p = _sc_degree(dst_r, n_pad)

  y0, comb0, res0, dinv = _stage_a(
      xp, Wb0, Wc0, bc0r, Wr0, row(br0), deg_p[0], deg_p[1], None, n_pad,
      first=True)
  s_p0 = _sc_edge_sum(y0, src_r, dst_r, n_pad)
  h1 = _stage_d(y0, s_p0[0], s_p0[1], comb0, res0, dinv, row(bias0), row(g0),
                row(be0), n_pad)

  y1, comb1, res1 = _stage_a(
      h1, Wb1, Wc1, bc1r, Wr1, row(br1), None, None, dinv, n_pad, first=False)
  s_p1 = _sc_edge_sum(y1, src_r, dst_r, n_pad)
  out = _stage_d_pool(y1, s_p1[0], s_p1[1], comb1, res1, dinv, row(bias1),
                      row(g1), row(be1), batch3, Wf, n_pad)
  return out


# speedup vs baseline: 2.2133x; 1.0657x over previous
import functools

import jax
import jax.numpy as jnp
from jax import lax
from jax.experimental import pallas as pl
from jax.experimental.pallas import tpu as pltpu
from jax.experimental.pallas import tpu_sc as plsc

NC = 2
NS = 16
NW = NC * NS
CH = 128
BLK = 256
NUM_GRAPHS = 64
H = 8
B = 4
F32 = jnp.float32


def _sc_mesh():
  return plsc.VectorSubcoreMesh(core_axis_name="c", subcore_axis_name="s")


def _sc_degree(dst_r, n_pad):
  nchunk = dst_r.shape[1]
  rpt = n_pad // NS

  @functools.partial(
      pl.kernel,
      out_type=jax.ShapeDtypeStruct((NC, n_pad, 16), F32),
      mesh=_sc_mesh(),
      compiler_params=pltpu.CompilerParams(use_tc_tiling_on_sc=False),
      scratch_types=[
          pltpu.VMEM((nchunk, CH), jnp.int32),
          pltpu.VMEM((CH, 16), F32),
          pltpu.VMEM((16, 16), F32),
          pltpu.VMEM_SHARED((n_pad, 16), F32),
      ],
  )
  def deg_kernel(dst_hbm, out_hbm, dst_v, ones_v, zero_v, acc):
    cid = lax.axis_index("c")
    sid = lax.axis_index("s")
    wid = sid * NC + cid
    pltpu.sync_copy(dst_hbm.at[wid], dst_v)
    for i in range(16):
      zero_v[i, :] = jnp.zeros((16,), F32)

    def fill(i, carry):
      ones_v[i, :] = jnp.ones((16,), F32)
      return carry

    lax.fori_loop(0, CH, fill, 0)
    base = sid * rpt
    for k in range(rpt // 16):
      pltpu.sync_copy(zero_v, acc.at[pl.ds(base + k * 16, 16)])
    plsc.subcore_barrier()

    def step(j, carry):
      pltpu.sync_copy(ones_v, acc.at[dst_v.at[j]], add=True)
      return carry

    lax.fori_loop(0, nchunk, step, 0)
    plsc.subcore_barrier()
    pltpu.sync_copy(acc.at[pl.ds(base, rpt)], out_hbm.at[cid, pl.ds(base, rpt)])

  return deg_kernel(dst_r)


def _sc_edge_sum(y, src_r, dst_r, n_pad):
  nchunk = src_r.shape[1]
  rpt = n_pad // NS

  @functools.partial(
      pl.kernel,
      out_type=jax.ShapeDtypeStruct((NC, n_pad, 64), F32),
      mesh=_sc_mesh(),
      compiler_params=pltpu.CompilerParams(use_tc_tiling_on_sc=False),
      scratch_types=[
          pltpu.VMEM((nchunk, CH), jnp.int32),
          pltpu.VMEM((nchunk, CH), jnp.int32),
          pltpu.VMEM((CH, 64), F32),
          pltpu.VMEM((CH, 64), F32),
          pltpu.VMEM((16, 64), F32),
          pltpu.VMEM_SHARED((n_pad, 64), F32),
          pltpu.SemaphoreType.DMA,
          pltpu.SemaphoreType.DMA,
      ],
  )
  def scat_kernel(y_hbm, src_hbm, dst_hbm, out_hbm, src_v, dst_v, rows0,
                  rows1, zero_v, acc, sem0, sem1):
    cid = lax.axis_index("c")
    sid = lax.axis_index("s")
    wid = sid * NC + cid
    pltpu.sync_copy(src_hbm.at[wid], src_v)
    pltpu.sync_copy(dst_hbm.at[wid], dst_v)
    for i in range(16):
      for j in range(4):
        zero_v[i, pl.ds(j * 16, 16)] = jnp.zeros((16,), F32)
    base = sid * rpt
    for k in range(rpt // 16):
      pltpu.sync_copy(zero_v, acc.at[pl.ds(base + k * 16, 16)])
    plsc.subcore_barrier()

    pltpu.async_copy(y_hbm.at[src_v.at[0]], rows0, sem0)

    def pair(i, carry):
      j = 2 * i
      pltpu.async_copy(y_hbm.at[src_v.at[j + 1]], rows1, sem1)
      pltpu.make_async_copy(y_hbm.at[pl.ds(0, CH)], rows0, sem0).wait()
      pltpu.sync_copy(rows0, acc.at[dst_v.at[j]], add=True)
      pltpu.async_copy(y_hbm.at[src_v.at[j + 2]], rows0, sem0)
      pltpu.make_async_copy(y_hbm.at[pl.ds(0, CH)], rows1, sem1).wait()
      pltpu.sync_copy(rows1, acc.at[dst_v.at[j + 1]], add=True)
      return carry

    lax.fori_loop(0, (nchunk - 1) // 2, pair, 0)
    pltpu.make_async_copy(y_hbm.at[pl.ds(0, CH)], rows0, sem0).wait()
    pltpu.sync_copy(rows0, acc.at[dst_v.at[nchunk - 1]], add=True)
    plsc.subcore_barrier()
    pltpu.sync_copy(acc.at[pl.ds(base, rpt)], out_hbm.at[cid, pl.ds(base, rpt)])

  return scat_kernel(y, src_r, dst_r)


def _stage_a(h, wb, wc, bc, wr, br, deg0, deg1, dinv_in, n_pad, first):
  grid = n_pad // BLK
  hid = h.shape[1]

  def body(*refs):
    if first:
      (h_ref, wb_ref, wc_ref, bc_ref, wr_ref, br_ref, d0_ref, d1_ref,
       y_ref, comb_ref, res_ref, dinv_ref) = refs
      deg = d0_ref[:, 0:1] + d1_ref[:, 0:1] + 1.0
      dinv = lax.rsqrt(deg)
      dinv_ref[...] = jnp.broadcast_to(dinv, (BLK, 16))
    else:
      (h_ref, wb_ref, wc_ref, bc_ref, wr_ref, br_ref, dv_ref,
       y_ref, comb_ref, res_ref) = refs
      dinv = dv_ref[:, 0:1]
    hb = h_ref[...]
    y_ref[...] = jnp.dot(hb, wb_ref[...], preferred_element_type=F32) * dinv
    comb_ref[...] = jnp.dot(hb, wc_ref[...],
                            preferred_element_type=F32) + bc_ref[...]
    res_ref[...] = jnp.dot(hb, wr_ref[...],
                           preferred_element_type=F32) + br_ref[...]

  in_specs = [
      pl.BlockSpec((BLK, hid), lambda i: (i, 0)),
      pl.BlockSpec((hid, 64), lambda i: (0, 0)),
      pl.BlockSpec((hid, 32), lambda i: (0, 0)),
      pl.BlockSpec((1, 32), lambda i: (0, 0)),
      pl.BlockSpec((hid, 128), lambda i: (0, 0)),
      pl.BlockSpec((1, 128), lambda i: (0, 0)),
  ]
  out_shape = [
      jax.ShapeDtypeStruct((n_pad, 64), F32),
      jax.ShapeDtypeStruct((n_pad, 32), F32),
      jax.ShapeDtypeStruct((n_pad, 128), F32),
  ]
  out_specs = [
      pl.BlockSpec((BLK, 64), lambda i: (i, 0)),
      pl.BlockSpec((BLK, 32), lambda i: (i, 0)),
      pl.BlockSpec((BLK, 128), lambda i: (i, 0)),
  ]
  if first:
    in_specs += [pl.BlockSpec((BLK, 16), lambda i: (i, 0)),
                 pl.BlockSpec((BLK, 16), lambda i: (i, 0))]
    out_shape.append(jax.ShapeDtypeStruct((n_pad, 16), F32))
    out_specs.append(pl.BlockSpec((BLK, 16), lambda i: (i, 0)))
    args = (h, wb, wc, bc, wr, br, deg0, deg1)
  else:
    in_specs.append(pl.BlockSpec((BLK, 16), lambda i: (i, 0)))
    args = (h, wb, wc, bc, wr, br, dinv_in)

  return pl.pallas_call(
      body, grid=(grid,), in_specs=in_specs, out_specs=out_specs,
      out_shape=out_shape)(*args)


def _combine_body(y_ref, s0_ref, s1_ref, comb_ref, res_ref, dv_ref, bias_ref,
                  g_ref, be_ref):
  dinv = dv_ref[:, 0:1]
  agg = (y_ref[...] + s0_ref[...] + s1_ref[...]) * dinv
  comb = comb_ref[...]
  conv = jnp.zeros((BLK, 128), F32)
  ri_e = lax.broadcasted_iota(jnp.int32, (32, 128), 0)
  ci_e = lax.broadcasted_iota(jnp.int32, (32, 128), 1)
  ri_f = lax.broadcasted_iota(jnp.int32, (64, 128), 0)
  ci_f = lax.broadcasted_iota(jnp.int32, (64, 128), 1)
  for b in range(B):
    eb = jnp.where((ri_e % B == b) & (ci_e // 16 == ri_e // B), 1.0, 0.0)
    fb = jnp.where((ri_f // 16 == b) & (ci_f % 16 == ri_f % 16), 1.0, 0.0)
    ce = jnp.dot(comb, eb, preferred_element_type=F32)
    ae = jnp.dot(agg, fb, preferred_element_type=F32)
    conv = conv + ce * ae
  o = conv + bias_ref[...] + res_ref[...]
  mu = jnp.mean(o, axis=1, keepdims=True)
  var = jnp.mean((o - mu) ** 2, axis=1, keepdims=True)
  hn = (o - mu) / jnp.sqrt(var + 1e-5) * g_ref[...] + be_ref[...]
  return jnp.maximum(hn, 0.0)


def _stage_d_a(y, s0, s1, comb, res, dinv, bias, g, be, wb, wc, bc, wr, br,
               n_pad):
  grid = n_pad // BLK

  def body(y_ref, s0_ref, s1_ref, comb_ref, res_ref, dv_ref, bias_ref, g_ref,
           be_ref, wb_ref, wc_ref, bc_ref, wr_ref, br_ref,
           y1_ref, comb1_ref, res1_ref):
    h1 = _combine_body(y_ref, s0_ref, s1_ref, comb_ref, res_ref, dv_ref,
                       bias_ref, g_ref, be_ref)
    dinv = dv_ref[:, 0:1]
    y1_ref[...] = jnp.dot(h1, wb_ref[...], preferred_element_type=F32) * dinv
    comb1_ref[...] = jnp.dot(h1, wc_ref[...],
                             preferred_element_type=F32) + bc_ref[...]
    res1_ref[...] = jnp.dot(h1, wr_ref[...],
                            preferred_element_type=F32) + br_ref[...]

  return pl.pallas_call(
      body, grid=(grid,),
      in_specs=[
          pl.BlockSpec((BLK, 64), lambda i: (i, 0)),
          pl.BlockSpec((BLK, 64), lambda i: (i, 0)),
          pl.BlockSpec((BLK, 64), lambda i: (i, 0)),
          pl.BlockSpec((BLK, 32), lambda i: (i, 0)),
          pl.BlockSpec((BLK, 128), lambda i: (i, 0)),
          pl.BlockSpec((BLK, 16), lambda i: (i, 0)),
          pl.BlockSpec((1, 128), lambda i: (0, 0)),
          pl.BlockSpec((1, 128), lambda i: (0, 0)),
          pl.BlockSpec((1, 128), lambda i: (0, 0)),
          pl.BlockSpec((128, 64), lambda i: (0, 0)),
          pl.BlockSpec((128, 32), lambda i: (0, 0)),
          pl.BlockSpec((1, 32), lambda i: (0, 0)),
          pl.BlockSpec((128, 128), lambda i: (0, 0)),
          pl.BlockSpec((1, 128), lambda i: (0, 0)),
      ],
      out_specs=[
          pl.BlockSpec((BLK, 64), lambda i: (i, 0)),
          pl.BlockSpec((BLK, 32), lambda i: (i, 0)),
          pl.BlockSpec((BLK, 128), lambda i: (i, 0)),
      ],
      out_shape=[
          jax.ShapeDtypeStruct((n_pad, 64), F32),
          jax.ShapeDtypeStruct((n_pad, 32), F32),
          jax.ShapeDtypeStruct((n_pad, 128), F32),
      ],
  )(y, s0, s1, comb, res, dinv, bias, g, be, wb, wc, bc, wr, br)


def _stage_d_pool(y, s0, s1, comb, res, dinv, bias, g, be, batch3, wf, n_pad):
  grid = n_pad // BLK

  def body(y_ref, s0_ref, s1_ref, comb_ref, res_ref, dv_ref, bias_ref, g_ref,
           be_ref, batch_ref, wf_ref, out_ref, sums, cnts):
    i = pl.program_id(0)

    @pl.when(i == 0)
    def _init():
      sums[...] = jnp.zeros((NUM_GRAPHS, 128), F32)
      cnts[...] = jnp.zeros((NUM_GRAPHS, 128), F32)

    h2 = _combine_body(y_ref, s0_ref, s1_ref, comb_ref, res_ref, dv_ref,
                       bias_ref, g_ref, be_ref)
    brow = batch_ref[0]
    gi = lax.broadcasted_iota(jnp.int32, (NUM_GRAPHS, BLK), 0)
    oh = jnp.where(gi == brow, 1.0, 0.0)
    sums[...] += jnp.dot(oh, h2, preferred_element_type=F32)
    cnts[...] += jnp.broadcast_to(
        jnp.sum(oh, axis=1, keepdims=True), (NUM_GRAPHS, 128))

    @pl.when(i == grid - 1)
    def _fin():
      pooled = sums[...] / jnp.maximum(cnts[...], 1.0)
      out_ref[...] = jnp.dot(pooled, wf_ref[...], preferred_element_type=F32)

  return pl.pallas_call(
      body, grid=(grid,),
      in_specs=[
          pl.BlockSpec((BLK, 64), lambda i: (i, 0)),
          pl.BlockSpec((BLK, 64), lambda i: (i, 0)),
          pl.BlockSpec((BLK, 64), lambda i: (i, 0)),
          pl.BlockSpec((BLK, 32), lambda i: (i, 0)),
          pl.BlockSpec((BLK, 128), lambda i: (i, 0)),
          pl.BlockSpec((BLK, 16), lambda i: (i, 0)),
          pl.BlockSpec((1, 128), lambda i: (0, 0)),
          pl.BlockSpec((1, 128), lambda i: (0, 0)),
          pl.BlockSpec((1, 128), lambda i: (0, 0)),
          pl.BlockSpec((1, 1, BLK), lambda i: (i, 0, 0)),
          pl.BlockSpec((128, 8), lambda i: (0, 0)),
      ],
      out_specs=pl.BlockSpec((NUM_GRAPHS, 8), lambda i: (0, 0)),
      out_shape=jax.ShapeDtypeStruct((NUM_GRAPHS, 8), F32),
      compiler_params=pltpu.CompilerParams(
          dimension_semantics=("arbitrary",)),
      scratch_shapes=[pltpu.VMEM((NUM_GRAPHS, 128), F32),
                      pltpu.VMEM((NUM_GRAPHS, 128), F32)],
  )(y, s0, s1, comb, res, dinv, bias, g, be, batch3, wf)


def kernel(x, edge_index, batch, Wb0, Wc0, bc0, bias0, Wr0, br0, g0, be0,
           Wb1, Wc1, bc1, bias1, Wr1, br1, g1, be1, Wf):
  n = x.shape[0]
  e = edge_index.shape[1]
  n_pad = ((n + BLK - 1) // BLK) * BLK
  nchunk = -(-e // (NW * CH))
  nchunk += 1 - nchunk % 2
  e_pad = NW * nchunk * CH

  xp = jnp.pad(x, ((0, n_pad - n), (0, 0)))
  batch_p = jnp.concatenate(
      [batch, jnp.full((n_pad - n,), NUM_GRAPHS, jnp.int32)])
  batch3 = batch_p.reshape(n_pad // BLK, 1, BLK)
  pad_e = n + jnp.arange(e_pad - e, dtype=jnp.int32) % (n_pad - n)
  src_r = jnp.concatenate([edge_index[0], pad_e]).reshape(NW, nchunk, CH)
  dst_r = jnp.concatenate([edge_index[1], pad_e]).reshape(NW, nchunk, CH)

  bc0r, bc1r = bc0.reshape(1, -1), bc1.reshape(1, -1)
  row = lambda v: v.reshape(1, -1)

  deg_p = _sc_degree(dst_r, n_pad)

  y0, comb0, res0, dinv = _stage_a(
      xp, Wb0, Wc0, bc0r, Wr0, row(br0), deg_p[0], deg_p[1], None, n_pad,
      first=True)
  s_p0 = _sc_edge_sum(y0, src_r, dst_r, n_pad)
  y1, comb1, res1 = _stage_d_a(
      y0, s_p0[0], s_p0[1], comb0, res0, dinv, row(bias0), row(g0), row(be0),
      Wb1, Wc1, bc1r, Wr1, row(br1), n_pad)
  s_p1 = _sc_edge_sum(y1, src_r, dst_r, n_pad)
  out = _stage_d_pool(y1, s_p1[0], s_p1[1], comb1, res1, dinv, row(bias1),
                      row(g1), row(be1), batch3, Wf, n_pad)
  return out


# speedup vs baseline: 2.4662x; 1.1142x over previous
import functools

import jax
import jax.numpy as jnp
from jax import lax
from jax.experimental import pallas as pl
from jax.experimental.pallas import tpu as pltpu
from jax.experimental.pallas import tpu_sc as plsc

NC = 2
NS = 16
NW = NC * NS
CH = 128
BLK = 512
NUM_GRAPHS = 64
H = 8
B = 4
F32 = jnp.float32


def _sc_mesh():
  return plsc.VectorSubcoreMesh(core_axis_name="c", subcore_axis_name="s")


def _sc_degree(dst_r, n_pad):
  nchunk = dst_r.shape[1]
  rpt = n_pad // NS

  @functools.partial(
      pl.kernel,
      out_type=jax.ShapeDtypeStruct((NC, n_pad, 16), F32),
      mesh=_sc_mesh(),
      compiler_params=pltpu.CompilerParams(use_tc_tiling_on_sc=False),
      scratch_types=[
          pltpu.VMEM((nchunk, CH), jnp.int32),
          pltpu.VMEM((CH, 16), F32),
          pltpu.VMEM((16, 16), F32),
          pltpu.VMEM_SHARED((n_pad, 16), F32),
      ],
  )
  def deg_kernel(dst_hbm, out_hbm, dst_v, ones_v, zero_v, acc):
    cid = lax.axis_index("c")
    sid = lax.axis_index("s")
    wid = sid * NC + cid
    pltpu.sync_copy(dst_hbm.at[wid], dst_v)
    for i in range(16):
      zero_v[i, :] = jnp.zeros((16,), F32)

    def fill(i, carry):
      ones_v[i, :] = jnp.ones((16,), F32)
      return carry

    lax.fori_loop(0, CH, fill, 0)
    base = sid * rpt
    for k in range(rpt // 16):
      pltpu.sync_copy(zero_v, acc.at[pl.ds(base + k * 16, 16)])
    plsc.subcore_barrier()

    def step(j, carry):
      pltpu.sync_copy(ones_v, acc.at[dst_v.at[j]], add=True)
      return carry

    lax.fori_loop(0, nchunk, step, 0)
    plsc.subcore_barrier()
    pltpu.sync_copy(acc.at[pl.ds(base, rpt)], out_hbm.at[cid, pl.ds(base, rpt)])

  return deg_kernel(dst_r)


def _sc_edge_sum(y, src_r, dst_r, n_pad):
  nchunk = src_r.shape[1]
  rpt = n_pad // NS

  @functools.partial(
      pl.kernel,
      out_type=jax.ShapeDtypeStruct((NC, n_pad, 64), F32),
      mesh=_sc_mesh(),
      compiler_params=pltpu.CompilerParams(use_tc_tiling_on_sc=False),
      scratch_types=[
          pltpu.VMEM((nchunk, CH), jnp.int32),
          pltpu.VMEM((nchunk, CH), jnp.int32),
          pltpu.VMEM((CH, 64), F32),
          pltpu.VMEM((CH, 64), F32),
          pltpu.VMEM((16, 64), F32),
          pltpu.VMEM_SHARED((n_pad, 64), F32),
          pltpu.SemaphoreType.DMA,
          pltpu.SemaphoreType.DMA,
      ],
  )
  def scat_kernel(y_hbm, src_hbm, dst_hbm, out_hbm, src_v, dst_v, rows0,
                  rows1, zero_v, acc, sem0, sem1):
    cid = lax.axis_index("c")
    sid = lax.axis_index("s")
    wid = sid * NC + cid
    pltpu.sync_copy(src_hbm.at[wid], src_v)
    pltpu.sync_copy(dst_hbm.at[wid], dst_v)
    for i in range(16):
      for j in range(4):
        zero_v[i, pl.ds(j * 16, 16)] = jnp.zeros((16,), F32)
    base = sid * rpt
    for k in range(rpt // 16):
      pltpu.sync_copy(zero_v, acc.at[pl.ds(base + k * 16, 16)])
    plsc.subcore_barrier()

    pltpu.async_copy(y_hbm.at[src_v.at[0]], rows0, sem0)

    def pair(i, carry):
      j = 2 * i
      pltpu.async_copy(y_hbm.at[src_v.at[j + 1]], rows1, sem1)
      pltpu.make_async_copy(y_hbm.at[pl.ds(0, CH)], rows0, sem0).wait()
      pltpu.sync_copy(rows0, acc.at[dst_v.at[j]], add=True)
      pltpu.async_copy(y_hbm.at[src_v.at[j + 2]], rows0, sem0)
      pltpu.make_async_copy(y_hbm.at[pl.ds(0, CH)], rows1, sem1).wait()
      pltpu.sync_copy(rows1, acc.at[dst_v.at[j + 1]], add=True)
      return carry

    lax.fori_loop(0, (nchunk - 1) // 2, pair, 0)
    pltpu.make_async_copy(y_hbm.at[pl.ds(0, CH)], rows0, sem0).wait()
    pltpu.sync_copy(rows0, acc.at[dst_v.at[nchunk - 1]], add=True)
    plsc.subcore_barrier()
    pltpu.sync_copy(acc.at[pl.ds(base, rpt)], out_hbm.at[cid, pl.ds(base, rpt)])

  return scat_kernel(y, src_r, dst_r)


def _stage_a(h, wb, wc, bc, wr, br, deg0, deg1, dinv_in, n_pad, first):
  grid = n_pad // BLK
  hid = h.shape[1]

  def body(*refs):
    if first:
      (h_ref, wb_ref, wc_ref, bc_ref, wr_ref, br_ref, d0_ref, d1_ref,
       y_ref, comb_ref, res_ref, dinv_ref) = refs
      deg = d0_ref[:, 0:1] + d1_ref[:, 0:1] + 1.0
      dinv = lax.rsqrt(deg)
      dinv_ref[...] = jnp.broadcast_to(dinv, (BLK, 16))
    else:
      (h_ref, wb_ref, wc_ref, bc_ref, wr_ref, br_ref, dv_ref,
       y_ref, comb_ref, res_ref) = refs
      dinv = dv_ref[:, 0:1]
    hb = h_ref[...]
    y_ref[...] = jnp.dot(hb, wb_ref[...], preferred_element_type=F32) * dinv
    comb_ref[...] = jnp.dot(hb, wc_ref[...],
                            preferred_element_type=F32) + bc_ref[...]
    res_ref[...] = jnp.dot(hb, wr_ref[...],
                           preferred_element_type=F32) + br_ref[...]

  in_specs = [
      pl.BlockSpec((BLK, hid), lambda i: (i, 0)),
      pl.BlockSpec((hid, 64), lambda i: (0, 0)),
      pl.BlockSpec((hid, 32), lambda i: (0, 0)),
      pl.BlockSpec((1, 32), lambda i: (0, 0)),
      pl.BlockSpec((hid, 128), lambda i: (0, 0)),
      pl.BlockSpec((1, 128), lambda i: (0, 0)),
  ]
  out_shape = [
      jax.ShapeDtypeStruct((n_pad, 64), F32),
      jax.ShapeDtypeStruct((n_pad, 32), F32),
      jax.ShapeDtypeStruct((n_pad, 128), F32),
  ]
  out_specs = [
      pl.BlockSpec((BLK, 64), lambda i: (i, 0)),
      pl.BlockSpec((BLK, 32), lambda i: (i, 0)),
      pl.BlockSpec((BLK, 128), lambda i: (i, 0)),
  ]
  if first:
    in_specs += [pl.BlockSpec((BLK, 16), lambda i: (i, 0)),
                 pl.BlockSpec((BLK, 16), lambda i: (i, 0))]
    out_shape.append(jax.ShapeDtypeStruct((n_pad, 16), F32))
    out_specs.append(pl.BlockSpec((BLK, 16), lambda i: (i, 0)))
    args = (h, wb, wc, bc, wr, br, deg0, deg1)
  else:
    in_specs.append(pl.BlockSpec((BLK, 16), lambda i: (i, 0)))
    args = (h, wb, wc, bc, wr, br, dinv_in)

  return pl.pallas_call(
      body, grid=(grid,), in_specs=in_specs, out_specs=out_specs,
      out_shape=out_shape)(*args)


def _combine_body(y_ref, s0_ref, s1_ref, comb_ref, res_ref, dv_ref, bias_ref,
                  g_ref, be_ref):
  dinv = dv_ref[:, 0:1]
  agg = (y_ref[...] + s0_ref[...] + s1_ref[...]) * dinv
  comb = comb_ref[...]
  conv = jnp.zeros((BLK, 128), F32)
  ri_e = lax.broadcasted_iota(jnp.int32, (32, 128), 0)
  ci_e = lax.broadcasted_iota(jnp.int32, (32, 128), 1)
  ri_f = lax.broadcasted_iota(jnp.int32, (64, 128), 0)
  ci_f = lax.broadcasted_iota(jnp.int32, (64, 128), 1)
  for b in range(B):
    eb = jnp.where((ri_e % B == b) & (ci_e // 16 == ri_e // B), 1.0, 0.0)
    fb = jnp.where((ri_f // 16 == b) & (ci_f % 16 == ri_f % 16), 1.0, 0.0)
    ce = jnp.dot(comb, eb, preferred_element_type=F32)
    ae = jnp.dot(agg, fb, preferred_element_type=F32)
    conv = conv + ce * ae
  o = conv + bias_ref[...] + res_ref[...]
  mu = jnp.mean(o, axis=1, keepdims=True)
  var = jnp.mean((o - mu) ** 2, axis=1, keepdims=True)
  hn = (o - mu) / jnp.sqrt(var + 1e-5) * g_ref[...] + be_ref[...]
  return jnp.maximum(hn, 0.0)


def _stage_d_a(y, s0, s1, comb, res, dinv, bias, g, be, wb, wc, bc, wr, br,
               n_pad):
  grid = n_pad // BLK

  def body(y_ref, s0_ref, s1_ref, comb_ref, res_ref, dv_ref, bias_ref, g_ref,
           be_ref, wb_ref, wc_ref, bc_ref, wr_ref, br_ref,
           y1_ref, comb1_ref, res1_ref):
    h1 = _combine_body(y_ref, s0_ref, s1_ref, comb_ref, res_ref, dv_ref,
                       bias_ref, g_ref, be_ref)
    dinv = dv_ref[:, 0:1]
    y1_ref[...] = jnp.dot(h1, wb_ref[...], preferred_element_type=F32) * dinv
    comb1_ref[...] = jnp.dot(h1, wc_ref[...],
                             preferred_element_type=F32) + bc_ref[...]
    res1_ref[...] = jnp.dot(h1, wr_ref[...],
                            preferred_element_type=F32) + br_ref[...]

  return pl.pallas_call(
      body, grid=(grid,),
      in_specs=[
          pl.BlockSpec((BLK, 64), lambda i: (i, 0)),
          pl.BlockSpec((BLK, 64), lambda i: (i, 0)),
          pl.BlockSpec((BLK, 64), lambda i: (i, 0)),
          pl.BlockSpec((BLK, 32), lambda i: (i, 0)),
          pl.BlockSpec((BLK, 128), lambda i: (i, 0)),
          pl.BlockSpec((BLK, 16), lambda i: (i, 0)),
          pl.BlockSpec((1, 128), lambda i: (0, 0)),
          pl.BlockSpec((1, 128), lambda i: (0, 0)),
          pl.BlockSpec((1, 128), lambda i: (0, 0)),
          pl.BlockSpec((128, 64), lambda i: (0, 0)),
          pl.BlockSpec((128, 32), lambda i: (0, 0)),
          pl.BlockSpec((1, 32), lambda i: (0, 0)),
          pl.BlockSpec((128, 128), lambda i: (0, 0)),
          pl.BlockSpec((1, 128), lambda i: (0, 0)),
      ],
      out_specs=[
          pl.BlockSpec((BLK, 64), lambda i: (i, 0)),
          pl.BlockSpec((BLK, 32), lambda i: (i, 0)),
          pl.BlockSpec((BLK, 128), lambda i: (i, 0)),
      ],
      out_shape=[
          jax.ShapeDtypeStruct((n_pad, 64), F32),
          jax.ShapeDtypeStruct((n_pad, 32), F32),
          jax.ShapeDtypeStruct((n_pad, 128), F32),
      ],
  )(y, s0, s1, comb, res, dinv, bias, g, be, wb, wc, bc, wr, br)


def _stage_d_pool(y, s0, s1, comb, res, dinv, bias, g, be, batch3, wf, n_pad):
  grid = n_pad // BLK

  def body(y_ref, s0_ref, s1_ref, comb_ref, res_ref, dv_ref, bias_ref, g_ref,
           be_ref, batch_ref, wf_ref, out_ref, sums, cnts):
    i = pl.program_id(0)

    @pl.when(i == 0)
    def _init():
      sums[...] = jnp.zeros((NUM_GRAPHS, 128), F32)
      cnts[...] = jnp.zeros((NUM_GRAPHS, 128), F32)

    h2 = _combine_body(y_ref, s0_ref, s1_ref, comb_ref, res_ref, dv_ref,
                       bias_ref, g_ref, be_ref)
    brow = batch_ref[0]
    gi = lax.broadcasted_iota(jnp.int32, (NUM_GRAPHS, BLK), 0)
    oh = jnp.where(gi == brow, 1.0, 0.0)
    sums[...] += jnp.dot(oh, h2, preferred_element_type=F32)
    cnts[...] += jnp.broadcast_to(
        jnp.sum(oh, axis=1, keepdims=True), (NUM_GRAPHS, 128))

    @pl.when(i == grid - 1)
    def _fin():
      pooled = sums[...] / jnp.maximum(cnts[...], 1.0)
      out_ref[...] = jnp.dot(pooled, wf_ref[...], preferred_element_type=F32)

  return pl.pallas_call(
      body, grid=(grid,),
      in_specs=[
          pl.BlockSpec((BLK, 64), lambda i: (i, 0)),
          pl.BlockSpec((BLK, 64), lambda i: (i, 0)),
          pl.BlockSpec((BLK, 64), lambda i: (i, 0)),
          pl.BlockSpec((BLK, 32), lambda i: (i, 0)),
          pl.BlockSpec((BLK, 128), lambda i: (i, 0)),
          pl.BlockSpec((BLK, 16), lambda i: (i, 0)),
          pl.BlockSpec((1, 128), lambda i: (0, 0)),
          pl.BlockSpec((1, 128), lambda i: (0, 0)),
          pl.BlockSpec((1, 128), lambda i: (0, 0)),
          pl.BlockSpec((1, 1, BLK), lambda i: (i, 0, 0)),
          pl.BlockSpec((128, 8), lambda i: (0, 0)),
      ],
      out_specs=pl.BlockSpec((NUM_GRAPHS, 8), lambda i: (0, 0)),
      out_shape=jax.ShapeDtypeStruct((NUM_GRAPHS, 8), F32),
      compiler_params=pltpu.CompilerParams(
          dimension_semantics=("arbitrary",)),
      scratch_shapes=[pltpu.VMEM((NUM_GRAPHS, 128), F32),
                      pltpu.VMEM((NUM_GRAPHS, 128), F32)],
  )(y, s0, s1, comb, res, dinv, bias, g, be, batch3, wf)


def kernel(x, edge_index, batch, Wb0, Wc0, bc0, bias0, Wr0, br0, g0, be0,
           Wb1, Wc1, bc1, bias1, Wr1, br1, g1, be1, Wf):
  n = x.shape[0]
  e = edge_index.shape[1]
  n_pad = ((n + BLK - 1) // BLK) * BLK
  nchunk = -(-e // (NW * CH))
  nchunk += 1 - nchunk % 2
  e_pad = NW * nchunk * CH

  xp = jnp.pad(x, ((0, n_pad - n), (0, 0)))
  batch_p = jnp.concatenate(
      [batch, jnp.full((n_pad - n,), NUM_GRAPHS, jnp.int32)])
  batch3 = batch_p.reshape(n_pad // BLK, 1, BLK)
  pad_e = n + jnp.arange(e_pad - e, dtype=jnp.int32) % (n_pad - n)
  src_r = jnp.concatenate([edge_index[0], pad_e]).reshape(NW, nchunk, CH)
  dst_r = jnp.concatenate([edge_index[1], pad_e]).reshape(NW, nchunk, CH)

  bc0r, bc1r = bc0.reshape(1, -1), bc1.reshape(1, -1)
  row = lambda v: v.reshape(1, -1)

  deg_p = _sc_degree(dst_r, n_pad)

  y0, comb0, res0, dinv = _stage_a(
      xp, Wb0, Wc0, bc0r, Wr0, row(br0), deg_p[0], deg_p[1], None, n_pad,
      first=True)
  s_p0 = _sc_edge_sum(y0, src_r, dst_r, n_pad)
  y1, comb1, res1 = _stage_d_a(
      y0, s_p0[0], s_p0[1], comb0, res0, dinv, row(bias0), row(g0), row(be0),
      Wb1, Wc1, bc1r, Wr1, row(br1), n_pad)
  s_p1 = _sc_edge_sum(y1, src_r, dst_r, n_pad)
  out = _stage_d_pool(y1, s_p1[0], s_p1[1], comb1, res1, dinv, row(bias1),
                      row(g1), row(be1), batch3, Wf, n_pad)
  return out


# speedup vs baseline: 2.6302x; 1.0665x over previous
import functools

import jax
import jax.numpy as jnp
from jax import lax
from jax.experimental import pallas as pl
from jax.experimental.pallas import tpu as pltpu
from jax.experimental.pallas import tpu_sc as plsc

NC = 2
NS = 16
NW = NC * NS
CH = 128
BLK = 1024
NUM_GRAPHS = 64
H = 8
B = 4
F32 = jnp.float32


def _sc_mesh():
  return plsc.VectorSubcoreMesh(core_axis_name="c", subcore_axis_name="s")


def _sc_degree(dst_r, n_pad):
  nchunk = dst_r.shape[1]
  rpt = n_pad // NS

  @functools.partial(
      pl.kernel,
      out_type=jax.ShapeDtypeStruct((NC, n_pad, 16), F32),
      mesh=_sc_mesh(),
      compiler_params=pltpu.CompilerParams(use_tc_tiling_on_sc=False),
      scratch_types=[
          pltpu.VMEM((nchunk, CH), jnp.int32),
          pltpu.VMEM((CH, 16), F32),
          pltpu.VMEM((16, 16), F32),
          pltpu.VMEM_SHARED((n_pad, 16), F32),
      ],
  )
  def deg_kernel(dst_hbm, out_hbm, dst_v, ones_v, zero_v, acc):
    cid = lax.axis_index("c")
    sid = lax.axis_index("s")
    wid = sid * NC + cid
    pltpu.sync_copy(dst_hbm.at[wid], dst_v)
    for i in range(16):
      zero_v[i, :] = jnp.zeros((16,), F32)

    def fill(i, carry):
      ones_v[i, :] = jnp.ones((16,), F32)
      return carry

    lax.fori_loop(0, CH, fill, 0)
    base = sid * rpt
    for k in range(rpt // 16):
      pltpu.sync_copy(zero_v, acc.at[pl.ds(base + k * 16, 16)])
    plsc.subcore_barrier()

    def step(j, carry):
      pltpu.sync_copy(ones_v, acc.at[dst_v.at[j]], add=True)
      return carry

    lax.fori_loop(0, nchunk, step, 0)
    plsc.subcore_barrier()
    pltpu.sync_copy(acc.at[pl.ds(base, rpt)], out_hbm.at[cid, pl.ds(base, rpt)])

  return deg_kernel(dst_r)


def _sc_edge_sum(y, src_r, dst_r, n_pad):
  nchunk = src_r.shape[1]
  rpt = n_pad // NS

  @functools.partial(
      pl.kernel,
      out_type=jax.ShapeDtypeStruct((NC, n_pad, 64), F32),
      mesh=_sc_mesh(),
      compiler_params=pltpu.CompilerParams(use_tc_tiling_on_sc=False),
      scratch_types=[
          pltpu.VMEM((nchunk, CH), jnp.int32),
          pltpu.VMEM((nchunk, CH), jnp.int32),
          pltpu.VMEM((CH, 64), F32),
          pltpu.VMEM((CH, 64), F32),
          pltpu.VMEM((16, 64), F32),
          pltpu.VMEM_SHARED((n_pad, 64), F32),
          pltpu.SemaphoreType.DMA,
          pltpu.SemaphoreType.DMA,
      ],
  )
  def scat_kernel(y_hbm, src_hbm, dst_hbm, out_hbm, src_v, dst_v, rows0,
                  rows1, zero_v, acc, sem0, sem1):
    cid = lax.axis_index("c")
    sid = lax.axis_index("s")
    wid = sid * NC + cid
    pltpu.sync_copy(src_hbm.at[wid], src_v)
    pltpu.sync_copy(dst_hbm.at[wid], dst_v)
    for i in range(16):
      for j in range(4):
        zero_v[i, pl.ds(j * 16, 16)] = jnp.zeros((16,), F32)
    base = sid * rpt
    for k in range(rpt // 16):
      pltpu.sync_copy(zero_v, acc.at[pl.ds(base + k * 16, 16)])
    plsc.subcore_barrier()

    pltpu.async_copy(y_hbm.at[src_v.at[0]], rows0, sem0)

    def pair(i, carry):
      j = 2 * i
      pltpu.async_copy(y_hbm.at[src_v.at[j + 1]], rows1, sem1)
      pltpu.make_async_copy(y_hbm.at[pl.ds(0, CH)], rows0, sem0).wait()
      pltpu.sync_copy(rows0, acc.at[dst_v.at[j]], add=True)
      pltpu.async_copy(y_hbm.at[src_v.at[j + 2]], rows0, sem0)
      pltpu.make_async_copy(y_hbm.at[pl.ds(0, CH)], rows1, sem1).wait()
      pltpu.sync_copy(rows1, acc.at[dst_v.at[j + 1]], add=True)
      return carry

    lax.fori_loop(0, (nchunk - 1) // 2, pair, 0)
    pltpu.make_async_copy(y_hbm.at[pl.ds(0, CH)], rows0, sem0).wait()
    pltpu.sync_copy(rows0, acc.at[dst_v.at[nchunk - 1]], add=True)
    plsc.subcore_barrier()
    pltpu.sync_copy(acc.at[pl.ds(base, rpt)], out_hbm.at[cid, pl.ds(base, rpt)])

  return scat_kernel(y, src_r, dst_r)


def _stage_a(h, wb, wc, bc, wr, br, deg0, deg1, dinv_in, n_pad, first):
  grid = n_pad // BLK
  hid = h.shape[1]

  def body(*refs):
    if first:
      (h_ref, wb_ref, wc_ref, bc_ref, wr_ref, br_ref, d0_ref, d1_ref,
       y_ref, comb_ref, res_ref, dinv_ref) = refs
      deg = d0_ref[:, 0:1] + d1_ref[:, 0:1] + 1.0
      dinv = lax.rsqrt(deg)
      dinv_ref[...] = jnp.broadcast_to(dinv, (BLK, 16))
    else:
      (h_ref, wb_ref, wc_ref, bc_ref, wr_ref, br_ref, dv_ref,
       y_ref, comb_ref, res_ref) = refs
      dinv = dv_ref[:, 0:1]
    hb = h_ref[...]
    y_ref[...] = jnp.dot(hb, wb_ref[...], preferred_element_type=F32) * dinv
    comb_ref[...] = jnp.dot(hb, wc_ref[...],
                            preferred_element_type=F32) + bc_ref[...]
    res_ref[...] = jnp.dot(hb, wr_ref[...],
                           preferred_element_type=F32) + br_ref[...]

  in_specs = [
      pl.BlockSpec((BLK, hid), lambda i: (i, 0)),
      pl.BlockSpec((hid, 64), lambda i: (0, 0)),
      pl.BlockSpec((hid, 32), lambda i: (0, 0)),
      pl.BlockSpec((1, 32), lambda i: (0, 0)),
      pl.BlockSpec((hid, 128), lambda i: (0, 0)),
      pl.BlockSpec((1, 128), lambda i: (0, 0)),
  ]
  out_shape = [
      jax.ShapeDtypeStruct((n_pad, 64), F32),
      jax.ShapeDtypeStruct((n_pad, 32), F32),
      jax.ShapeDtypeStruct((n_pad, 128), F32),
  ]
  out_specs = [
      pl.BlockSpec((BLK, 64), lambda i: (i, 0)),
      pl.BlockSpec((BLK, 32), lambda i: (i, 0)),
      pl.BlockSpec((BLK, 128), lambda i: (i, 0)),
  ]
  if first:
    in_specs += [pl.BlockSpec((BLK, 16), lambda i: (i, 0)),
                 pl.BlockSpec((BLK, 16), lambda i: (i, 0))]
    out_shape.append(jax.ShapeDtypeStruct((n_pad, 16), F32))
    out_specs.append(pl.BlockSpec((BLK, 16), lambda i: (i, 0)))
    args = (h, wb, wc, bc, wr, br, deg0, deg1)
  else:
    in_specs.append(pl.BlockSpec((BLK, 16), lambda i: (i, 0)))
    args = (h, wb, wc, bc, wr, br, dinv_in)

  return pl.pallas_call(
      body, grid=(grid,), in_specs=in_specs, out_specs=out_specs,
      out_shape=out_shape)(*args)


def _combine_body(y_ref, s0_ref, s1_ref, comb_ref, res_ref, dv_ref, bias_ref,
                  g_ref, be_ref):
  dinv = dv_ref[:, 0:1]
  agg = (y_ref[...] + s0_ref[...] + s1_ref[...]) * dinv
  comb = comb_ref[...]
  conv = jnp.zeros((BLK, 128), F32)
  ri_e = lax.broadcasted_iota(jnp.int32, (32, 128), 0)
  ci_e = lax.broadcasted_iota(jnp.int32, (32, 128), 1)
  ri_f = lax.broadcasted_iota(jnp.int32, (64, 128), 0)
  ci_f = lax.broadcasted_iota(jnp.int32, (64, 128), 1)
  for b in range(B):
    eb = jnp.where((ri_e % B == b) & (ci_e // 16 == ri_e // B), 1.0, 0.0)
    fb = jnp.where((ri_f // 16 == b) & (ci_f % 16 == ri_f % 16), 1.0, 0.0)
    ce = jnp.dot(comb, eb, preferred_element_type=F32)
    ae = jnp.dot(agg, fb, preferred_element_type=F32)
    conv = conv + ce * ae
  o = conv + bias_ref[...] + res_ref[...]
  mu = jnp.mean(o, axis=1, keepdims=True)
  var = jnp.mean((o - mu) ** 2, axis=1, keepdims=True)
  hn = (o - mu) / jnp.sqrt(var + 1e-5) * g_ref[...] + be_ref[...]
  return jnp.maximum(hn, 0.0)


def _stage_d_a(y, s0, s1, comb, res, dinv, bias, g, be, wb, wc, bc, wr, br,
               n_pad):
  grid = n_pad // BLK

  def body(y_ref, s0_ref, s1_ref, comb_ref, res_ref, dv_ref, bias_ref, g_ref,
           be_ref, wb_ref, wc_ref, bc_ref, wr_ref, br_ref,
           y1_ref, comb1_ref, res1_ref):
    h1 = _combine_body(y_ref, s0_ref, s1_ref, comb_ref, res_ref, dv_ref,
                       bias_ref, g_ref, be_ref)
    dinv = dv_ref[:, 0:1]
    y1_ref[...] = jnp.dot(h1, wb_ref[...], preferred_element_type=F32) * dinv
    comb1_ref[...] = jnp.dot(h1, wc_ref[...],
                             preferred_element_type=F32) + bc_ref[...]
    res1_ref[...] = jnp.dot(h1, wr_ref[...],
                            preferred_element_type=F32) + br_ref[...]

  return pl.pallas_call(
      body, grid=(grid,),
      in_specs=[
          pl.BlockSpec((BLK, 64), lambda i: (i, 0)),
          pl.BlockSpec((BLK, 64), lambda i: (i, 0)),
          pl.BlockSpec((BLK, 64), lambda i: (i, 0)),
          pl.BlockSpec((BLK, 32), lambda i: (i, 0)),
          pl.BlockSpec((BLK, 128), lambda i: (i, 0)),
          pl.BlockSpec((BLK, 16), lambda i: (i, 0)),
          pl.BlockSpec((1, 128), lambda i: (0, 0)),
          pl.BlockSpec((1, 128), lambda i: (0, 0)),
          pl.BlockSpec((1, 128), lambda i: (0, 0)),
          pl.BlockSpec((128, 64), lambda i: (0, 0)),
          pl.BlockSpec((128, 32), lambda i: (0, 0)),
          pl.BlockSpec((1, 32), lambda i: (0, 0)),
          pl.BlockSpec((128, 128), lambda i: (0, 0)),
          pl.BlockSpec((1, 128), lambda i: (0, 0)),
      ],
      out_specs=[
          pl.BlockSpec((BLK, 64), lambda i: (i, 0)),
          pl.BlockSpec((BLK, 32), lambda i: (i, 0)),
          pl.BlockSpec((BLK, 128), lambda i: (i, 0)),
      ],
      out_shape=[
          jax.ShapeDtypeStruct((n_pad, 64), F32),
          jax.ShapeDtypeStruct((n_pad, 32), F32),
          jax.ShapeDtypeStruct((n_pad, 128), F32),
      ],
  )(y, s0, s1, comb, res, dinv, bias, g, be, wb, wc, bc, wr, br)


def _stage_d_pool(y, s0, s1, comb, res, dinv, bias, g, be, batch3, wf, n_pad):
  grid = n_pad // BLK

  def body(y_ref, s0_ref, s1_ref, comb_ref, res_ref, dv_ref, bias_ref, g_ref,
           be_ref, batch_ref, wf_ref, out_ref, sums, cnts):
    i = pl.program_id(0)

    @pl.when(i == 0)
    def _init():
      sums[...] = jnp.zeros((NUM_GRAPHS, 128), F32)
      cnts[...] = jnp.zeros((NUM_GRAPHS, 128), F32)

    h2 = _combine_body(y_ref, s0_ref, s1_ref, comb_ref, res_ref, dv_ref,
                       bias_ref, g_ref, be_ref)
    brow = batch_ref[0]
    gi = lax.broadcasted_iota(jnp.int32, (NUM_GRAPHS, BLK), 0)
    oh = jnp.where(gi == brow, 1.0, 0.0)
    sums[...] += jnp.dot(oh, h2, preferred_element_type=F32)
    cnts[...] += jnp.broadcast_to(
        jnp.sum(oh, axis=1, keepdims=True), (NUM_GRAPHS, 128))

    @pl.when(i == grid - 1)
    def _fin():
      pooled = sums[...] / jnp.maximum(cnts[...], 1.0)
      out_ref[...] = jnp.dot(pooled, wf_ref[...], preferred_element_type=F32)

  return pl.pallas_call(
      body, grid=(grid,),
      in_specs=[
          pl.BlockSpec((BLK, 64), lambda i: (i, 0)),
          pl.BlockSpec((BLK, 64), lambda i: (i, 0)),
          pl.BlockSpec((BLK, 64), lambda i: (i, 0)),
          pl.BlockSpec((BLK, 32), lambda i: (i, 0)),
          pl.BlockSpec((BLK, 128), lambda i: (i, 0)),
          pl.BlockSpec((BLK, 16), lambda i: (i, 0)),
          pl.BlockSpec((1, 128), lambda i: (0, 0)),
          pl.BlockSpec((1, 128), lambda i: (0, 0)),
          pl.BlockSpec((1, 128), lambda i: (0, 0)),
          pl.BlockSpec((1, 1, BLK), lambda i: (i, 0, 0)),
          pl.BlockSpec((128, 8), lambda i: (0, 0)),
      ],
      out_specs=pl.BlockSpec((NUM_GRAPHS, 8), lambda i: (0, 0)),
      out_shape=jax.ShapeDtypeStruct((NUM_GRAPHS, 8), F32),
      compiler_params=pltpu.CompilerParams(
          dimension_semantics=("arbitrary",)),
      scratch_shapes=[pltpu.VMEM((NUM_GRAPHS, 128), F32),
                      pltpu.VMEM((NUM_GRAPHS, 128), F32)],
  )(y, s0, s1, comb, res, dinv, bias, g, be, batch3, wf)


def kernel(x, edge_index, batch, Wb0, Wc0, bc0, bias0, Wr0, br0, g0, be0,
           Wb1, Wc1, bc1, bias1, Wr1, br1, g1, be1, Wf):
  n = x.shape[0]
  e = edge_index.shape[1]
  n_pad = ((n + BLK - 1) // BLK) * BLK
  nchunk = -(-e // (NW * CH))
  nchunk += 1 - nchunk % 2
  e_pad = NW * nchunk * CH

  xp = jnp.pad(x, ((0, n_pad - n), (0, 0)))
  batch_p = jnp.concatenate(
      [batch, jnp.full((n_pad - n,), NUM_GRAPHS, jnp.int32)])
  batch3 = batch_p.reshape(n_pad // BLK, 1, BLK)
  pad_e = n + jnp.arange(e_pad - e, dtype=jnp.int32) % (n_pad - n)
  src_r = jnp.concatenate([edge_index[0], pad_e]).reshape(NW, nchunk, CH)
  dst_r = jnp.concatenate([edge_index[1], pad_e]).reshape(NW, nchunk, CH)

  bc0r, bc1r = bc0.reshape(1, -1), bc1.reshape(1, -1)
  row = lambda v: v.reshape(1, -1)

  deg_p = _sc_degree(dst_r, n_pad)

  y0, comb0, res0, dinv = _stage_a(
      xp, Wb0, Wc0, bc0r, Wr0, row(br0), deg_p[0], deg_p[1], None, n_pad,
      first=True)
  s_p0 = _sc_edge_sum(y0, src_r, dst_r, n_pad)
  y1, comb1, res1 = _stage_d_a(
      y0, s_p0[0], s_p0[1], comb0, res0, dinv, row(bias0), row(g0), row(be0),
      Wb1, Wc1, bc1r, Wr1, row(br1), n_pad)
  s_p1 = _sc_edge_sum(y1, src_r, dst_r, n_pad)
  out = _stage_d_pool(y1, s_p1[0], s_p1[1], comb1, res1, dinv, row(bias1),
                      row(g1), row(be1), batch3, Wf, n_pad)
  return out


# speedup vs baseline: 2.6763x; 1.0175x over previous
import functools

import jax
import jax.numpy as jnp
from jax import lax
from jax.experimental import pallas as pl
from jax.experimental.pallas import tpu as pltpu
from jax.experimental.pallas import tpu_sc as plsc

NC = 2
NS = 16
NW = NC * NS
CH = 128
BLK = 2048
NUM_GRAPHS = 64
H = 8
B = 4
F32 = jnp.float32


def _sc_mesh():
  return plsc.VectorSubcoreMesh(core_axis_name="c", subcore_axis_name="s")


def _sc_degree(dst_r, n_pad):
  nchunk = dst_r.shape[1]
  rpt = n_pad // NS

  @functools.partial(
      pl.kernel,
      out_type=jax.ShapeDtypeStruct((NC, n_pad, 16), F32),
      mesh=_sc_mesh(),
      compiler_params=pltpu.CompilerParams(use_tc_tiling_on_sc=False),
      scratch_types=[
          pltpu.VMEM((nchunk, CH), jnp.int32),
          pltpu.VMEM((CH, 16), F32),
          pltpu.VMEM((16, 16), F32),
          pltpu.VMEM_SHARED((n_pad, 16), F32),
      ],
  )
  def deg_kernel(dst_hbm, out_hbm, dst_v, ones_v, zero_v, acc):
    cid = lax.axis_index("c")
    sid = lax.axis_index("s")
    wid = sid * NC + cid
    pltpu.sync_copy(dst_hbm.at[wid], dst_v)
    for i in range(16):
      zero_v[i, :] = jnp.zeros((16,), F32)

    def fill(i, carry):
      ones_v[i, :] = jnp.ones((16,), F32)
      return carry

    lax.fori_loop(0, CH, fill, 0)
    base = sid * rpt
    for k in range(rpt // 16):
      pltpu.sync_copy(zero_v, acc.at[pl.ds(base + k * 16, 16)])
    plsc.subcore_barrier()

    def step(j, carry):
      pltpu.sync_copy(ones_v, acc.at[dst_v.at[j]], add=True)
      return carry

    lax.fori_loop(0, nchunk, step, 0)
    plsc.subcore_barrier()
    pltpu.sync_copy(acc.at[pl.ds(base, rpt)], out_hbm.at[cid, pl.ds(base, rpt)])

  return deg_kernel(dst_r)


def _sc_edge_sum(y, src_r, dst_r, n_pad):
  nchunk = src_r.shape[1]
  rpt = n_pad // NS

  @functools.partial(
      pl.kernel,
      out_type=jax.ShapeDtypeStruct((NC, n_pad, 64), F32),
      mesh=_sc_mesh(),
      compiler_params=pltpu.CompilerParams(use_tc_tiling_on_sc=False),
      scratch_types=[
          pltpu.VMEM((nchunk, CH), jnp.int32),
          pltpu.VMEM((nchunk, CH), jnp.int32),
          pltpu.VMEM((CH, 64), F32),
          pltpu.VMEM((CH, 64), F32),
          pltpu.VMEM((16, 64), F32),
          pltpu.VMEM_SHARED((n_pad, 64), F32),
          pltpu.SemaphoreType.DMA,
          pltpu.SemaphoreType.DMA,
      ],
  )
  def scat_kernel(y_hbm, src_hbm, dst_hbm, out_hbm, src_v, dst_v, rows0,
                  rows1, zero_v, acc, sem0, sem1):
    cid = lax.axis_index("c")
    sid = lax.axis_index("s")
    wid = sid * NC + cid
    pltpu.sync_copy(src_hbm.at[wid], src_v)
    pltpu.sync_copy(dst_hbm.at[wid], dst_v)
    for i in range(16):
      for j in range(4):
        zero_v[i, pl.ds(j * 16, 16)] = jnp.zeros((16,), F32)
    base = sid * rpt
    for k in range(rpt // 16):
      pltpu.sync_copy(zero_v, acc.at[pl.ds(base + k * 16, 16)])
    plsc.subcore_barrier()

    pltpu.async_copy(y_hbm.at[src_v.at[0]], rows0, sem0)

    def pair(i, carry):
      j = 2 * i
      pltpu.async_copy(y_hbm.at[src_v.at[j + 1]], rows1, sem1)
      pltpu.make_async_copy(y_hbm.at[pl.ds(0, CH)], rows0, sem0).wait()
      pltpu.sync_copy(rows0, acc.at[dst_v.at[j]], add=True)
      pltpu.async_copy(y_hbm.at[src_v.at[j + 2]], rows0, sem0)
      pltpu.make_async_copy(y_hbm.at[pl.ds(0, CH)], rows1, sem1).wait()
      pltpu.sync_copy(rows1, acc.at[dst_v.at[j + 1]], add=True)
      return carry

    lax.fori_loop(0, (nchunk - 1) // 2, pair, 0)
    pltpu.make_async_copy(y_hbm.at[pl.ds(0, CH)], rows0, sem0).wait()
    pltpu.sync_copy(rows0, acc.at[dst_v.at[nchunk - 1]], add=True)
    plsc.subcore_barrier()
    pltpu.sync_copy(acc.at[pl.ds(base, rpt)], out_hbm.at[cid, pl.ds(base, rpt)])

  return scat_kernel(y, src_r, dst_r)


def _stage_a(h, wb, wc, bc, wr, br, deg0, deg1, dinv_in, n_pad, first):
  grid = n_pad // BLK
  hid = h.shape[1]

  def body(*refs):
    if first:
      (h_ref, wb_ref, wc_ref, bc_ref, wr_ref, br_ref, d0_ref, d1_ref,
       y_ref, comb_ref, res_ref, dinv_ref) = refs
      deg = d0_ref[:, 0:1] + d1_ref[:, 0:1] + 1.0
      dinv = lax.rsqrt(deg)
      dinv_ref[...] = jnp.broadcast_to(dinv, (BLK, 16))
    else:
      (h_ref, wb_ref, wc_ref, bc_ref, wr_ref, br_ref, dv_ref,
       y_ref, comb_ref, res_ref) = refs
      dinv = dv_ref[:, 0:1]
    hb = h_ref[...]
    y_ref[...] = jnp.dot(hb, wb_ref[...], preferred_element_type=F32) * dinv
    comb_ref[...] = jnp.dot(hb, wc_ref[...],
                            preferred_element_type=F32) + bc_ref[...]
    res_ref[...] = jnp.dot(hb, wr_ref[...],
                           preferred_element_type=F32) + br_ref[...]

  in_specs = [
      pl.BlockSpec((BLK, hid), lambda i: (i, 0)),
      pl.BlockSpec((hid, 64), lambda i: (0, 0)),
      pl.BlockSpec((hid, 32), lambda i: (0, 0)),
      pl.BlockSpec((1, 32), lambda i: (0, 0)),
      pl.BlockSpec((hid, 128), lambda i: (0, 0)),
      pl.BlockSpec((1, 128), lambda i: (0, 0)),
  ]
  out_shape = [
      jax.ShapeDtypeStruct((n_pad, 64), F32),
      jax.ShapeDtypeStruct((n_pad, 32), F32),
      jax.ShapeDtypeStruct((n_pad, 128), F32),
  ]
  out_specs = [
      pl.BlockSpec((BLK, 64), lambda i: (i, 0)),
      pl.BlockSpec((BLK, 32), lambda i: (i, 0)),
      pl.BlockSpec((BLK, 128), lambda i: (i, 0)),
  ]
  if first:
    in_specs += [pl.BlockSpec((BLK, 16), lambda i: (i, 0)),
                 pl.BlockSpec((BLK, 16), lambda i: (i, 0))]
    out_shape.append(jax.ShapeDtypeStruct((n_pad, 16), F32))
    out_specs.append(pl.BlockSpec((BLK, 16), lambda i: (i, 0)))
    args = (h, wb, wc, bc, wr, br, deg0, deg1)
  else:
    in_specs.append(pl.BlockSpec((BLK, 16), lambda i: (i, 0)))
    args = (h, wb, wc, bc, wr, br, dinv_in)

  return pl.pallas_call(
      body, grid=(grid,), in_specs=in_specs, out_specs=out_specs,
      out_shape=out_shape)(*args)


def _combine_body(y_ref, s0_ref, s1_ref, comb_ref, res_ref, dv_ref, bias_ref,
                  g_ref, be_ref):
  dinv = dv_ref[:, 0:1]
  agg = (y_ref[...] + s0_ref[...] + s1_ref[...]) * dinv
  comb = comb_ref[...]
  conv = jnp.zeros((BLK, 128), F32)
  ri_e = lax.broadcasted_iota(jnp.int32, (32, 128), 0)
  ci_e = lax.broadcasted_iota(jnp.int32, (32, 128), 1)
  ri_f = lax.broadcasted_iota(jnp.int32, (64, 128), 0)
  ci_f = lax.broadcasted_iota(jnp.int32, (64, 128), 1)
  for b in range(B):
    eb = jnp.where((ri_e % B == b) & (ci_e // 16 == ri_e // B), 1.0, 0.0)
    fb = jnp.where((ri_f // 16 == b) & (ci_f % 16 == ri_f % 16), 1.0, 0.0)
    ce = jnp.dot(comb, eb, preferred_element_type=F32)
    ae = jnp.dot(agg, fb, preferred_element_type=F32)
    conv = conv + ce * ae
  o = conv + bias_ref[...] + res_ref[...]
  mu = jnp.mean(o, axis=1, keepdims=True)
  var = jnp.mean((o - mu) ** 2, axis=1, keepdims=True)
  hn = (o - mu) / jnp.sqrt(var + 1e-5) * g_ref[...] + be_ref[...]
  return jnp.maximum(hn, 0.0)


def _stage_d_a(y, s0, s1, comb, res, dinv, bias, g, be, wb, wc, bc, wr, br,
               n_pad):
  grid = n_pad // BLK

  def body(y_ref, s0_ref, s1_ref, comb_ref, res_ref, dv_ref, bias_ref, g_ref,
           be_ref, wb_ref, wc_ref, bc_ref, wr_ref, br_ref,
           y1_ref, comb1_ref, res1_ref):
    h1 = _combine_body(y_ref, s0_ref, s1_ref, comb_ref, res_ref, dv_ref,
                       bias_ref, g_ref, be_ref)
    dinv = dv_ref[:, 0:1]
    y1_ref[...] = jnp.dot(h1, wb_ref[...], preferred_element_type=F32) * dinv
    comb1_ref[...] = jnp.dot(h1, wc_ref[...],
                             preferred_element_type=F32) + bc_ref[...]
    res1_ref[...] = jnp.dot(h1, wr_ref[...],
                            preferred_element_type=F32) + br_ref[...]

  return pl.pallas_call(
      body, grid=(grid,),
      in_specs=[
          pl.BlockSpec((BLK, 64), lambda i: (i, 0)),
          pl.BlockSpec((BLK, 64), lambda i: (i, 0)),
          pl.BlockSpec((BLK, 64), lambda i: (i, 0)),
          pl.BlockSpec((BLK, 32), lambda i: (i, 0)),
          pl.BlockSpec((BLK, 128), lambda i: (i, 0)),
          pl.BlockSpec((BLK, 16), lambda i: (i, 0)),
          pl.BlockSpec((1, 128), lambda i: (0, 0)),
          pl.BlockSpec((1, 128), lambda i: (0, 0)),
          pl.BlockSpec((1, 128), lambda i: (0, 0)),
          pl.BlockSpec((128, 64), lambda i: (0, 0)),
          pl.BlockSpec((128, 32), lambda i: (0, 0)),
          pl.BlockSpec((1, 32), lambda i: (0, 0)),
          pl.BlockSpec((128, 128), lambda i: (0, 0)),
          pl.BlockSpec((1, 128), lambda i: (0, 0)),
      ],
      out_specs=[
          pl.BlockSpec((BLK, 64), lambda i: (i, 0)),
          pl.BlockSpec((BLK, 32), lambda i: (i, 0)),
          pl.BlockSpec((BLK, 128), lambda i: (i, 0)),
      ],
      out_shape=[
          jax.ShapeDtypeStruct((n_pad, 64), F32),
          jax.ShapeDtypeStruct((n_pad, 32), F32),
          jax.ShapeDtypeStruct((n_pad, 128), F32),
      ],
  )(y, s0, s1, comb, res, dinv, bias, g, be, wb, wc, bc, wr, br)


def _stage_d_pool(y, s0, s1, comb, res, dinv, bias, g, be, batch3, wf, n_pad):
  grid = n_pad // BLK

  def body(y_ref, s0_ref, s1_ref, comb_ref, res_ref, dv_ref, bias_ref, g_ref,
           be_ref, batch_ref, wf_ref, out_ref, sums, cnts):
    i = pl.program_id(0)

    @pl.when(i == 0)
    def _init():
      sums[...] = jnp.zeros((NUM_GRAPHS, 128), F32)
      cnts[...] = jnp.zeros((NUM_GRAPHS, 128), F32)

    h2 = _combine_body(y_ref, s0_ref, s1_ref, comb_ref, res_ref, dv_ref,
                       bias_ref, g_ref, be_ref)
    brow = batch_ref[0]
    gi = lax.broadcasted_iota(jnp.int32, (NUM_GRAPHS, BLK), 0)
    oh = jnp.where(gi == brow, 1.0, 0.0)
    sums[...] += jnp.dot(oh, h2, preferred_element_type=F32)
    cnts[...] += jnp.broadcast_to(
        jnp.sum(oh, axis=1, keepdims=True), (NUM_GRAPHS, 128))

    @pl.when(i == grid - 1)
    def _fin():
      pooled = sums[...] / jnp.maximum(cnts[...], 1.0)
      out_ref[...] = jnp.dot(pooled, wf_ref[...], preferred_element_type=F32)

  return pl.pallas_call(
      body, grid=(grid,),
      in_specs=[
          pl.BlockSpec((BLK, 64), lambda i: (i, 0)),
          pl.BlockSpec((BLK, 64), lambda i: (i, 0)),
          pl.BlockSpec((BLK, 64), lambda i: (i, 0)),
          pl.BlockSpec((BLK, 32), lambda i: (i, 0)),
          pl.BlockSpec((BLK, 128), lambda i: (i, 0)),
          pl.BlockSpec((BLK, 16), lambda i: (i, 0)),
          pl.BlockSpec((1, 128), lambda i: (0, 0)),
          pl.BlockSpec((1, 128), lambda i: (0, 0)),
          pl.BlockSpec((1, 128), lambda i: (0, 0)),
          pl.BlockSpec((1, 1, BLK), lambda i: (i, 0, 0)),
          pl.BlockSpec((128, 8), lambda i: (0, 0)),
      ],
      out_specs=pl.BlockSpec((NUM_GRAPHS, 8), lambda i: (0, 0)),
      out_shape=jax.ShapeDtypeStruct((NUM_GRAPHS, 8), F32),
      compiler_params=pltpu.CompilerParams(
          dimension_semantics=("arbitrary",)),
      scratch_shapes=[pltpu.VMEM((NUM_GRAPHS, 128), F32),
                      pltpu.VMEM((NUM_GRAPHS, 128), F32)],
  )(y, s0, s1, comb, res, dinv, bias, g, be, batch3, wf)


def kernel(x, edge_index, batch, Wb0, Wc0, bc0, bias0, Wr0, br0, g0, be0,
           Wb1, Wc1, bc1, bias1, Wr1, br1, g1, be1, Wf):
  n = x.shape[0]
  e = edge_index.shape[1]
  n_pad = ((n + BLK - 1) // BLK) * BLK
  nchunk = -(-e // (NW * CH))
  nchunk += 1 - nchunk % 2
  e_pad = NW * nchunk * CH

  xp = jnp.pad(x, ((0, n_pad - n), (0, 0)))
  batch_p = jnp.concatenate(
      [batch, jnp.full((n_pad - n,), NUM_GRAPHS, jnp.int32)])
  batch3 = batch_p.reshape(n_pad // BLK, 1, BLK)
  pad_e = n + jnp.arange(e_pad - e, dtype=jnp.int32) % (n_pad - n)
  src_r = jnp.concatenate([edge_index[0], pad_e]).reshape(NW, nchunk, CH)
  dst_r = jnp.concatenate([edge_index[1], pad_e]).reshape(NW, nchunk, CH)

  bc0r, bc1r = bc0.reshape(1, -1), bc1.reshape(1, -1)
  row = lambda v: v.reshape(1, -1)

  deg_p = _sc_degree(dst_r, n_pad)

  y0, comb0, res0, dinv = _stage_a(
      xp, Wb0, Wc0, bc0r, Wr0, row(br0), deg_p[0], deg_p[1], None, n_pad,
      first=True)
  s_p0 = _sc_edge_sum(y0, src_r, dst_r, n_pad)
  y1, comb1, res1 = _stage_d_a(
      y0, s_p0[0], s_p0[1], comb0, res0, dinv, row(bias0), row(g0), row(be0),
      Wb1, Wc1, bc1r, Wr1, row(br1), n_pad)
  s_p1 = _sc_edge_sum(y1, src_r, dst_r, n_pad)
  out = _stage_d_pool(y1, s_p1[0], s_p1[1], comb1, res1, dinv, row(bias1),
                      row(g1), row(be1), batch3, Wf, n_pad)
  return out


# speedup vs baseline: 2.8485x; 1.0644x over previous
import functools

import jax
import jax.numpy as jnp
from jax import lax
from jax.experimental import pallas as pl
from jax.experimental.pallas import tpu as pltpu
from jax.experimental.pallas import tpu_sc as plsc

NC = 2
NS = 16
NW = NC * NS
CH = 256
BLK = 2048
NUM_GRAPHS = 64
H = 8
B = 4
F32 = jnp.float32


def _sc_mesh():
  return plsc.VectorSubcoreMesh(core_axis_name="c", subcore_axis_name="s")


def _sc_degree(dst_r, n_pad):
  nchunk = dst_r.shape[1]
  rpt = n_pad // NS

  @functools.partial(
      pl.kernel,
      out_type=jax.ShapeDtypeStruct((NC, n_pad, 16), F32),
      mesh=_sc_mesh(),
      compiler_params=pltpu.CompilerParams(use_tc_tiling_on_sc=False),
      scratch_types=[
          pltpu.VMEM((nchunk, CH), jnp.int32),
          pltpu.VMEM((CH, 16), F32),
          pltpu.VMEM((16, 16), F32),
          pltpu.VMEM_SHARED((n_pad, 16), F32),
      ],
  )
  def deg_kernel(dst_hbm, out_hbm, dst_v, ones_v, zero_v, acc):
    cid = lax.axis_index("c")
    sid = lax.axis_index("s")
    wid = sid * NC + cid
    pltpu.sync_copy(dst_hbm.at[wid], dst_v)
    for i in range(16):
      zero_v[i, :] = jnp.zeros((16,), F32)

    def fill(i, carry):
      ones_v[i, :] = jnp.ones((16,), F32)
      return carry

    lax.fori_loop(0, CH, fill, 0)
    base = sid * rpt
    for k in range(rpt // 16):
      pltpu.sync_copy(zero_v, acc.at[pl.ds(base + k * 16, 16)])
    plsc.subcore_barrier()

    def step(j, carry):
      pltpu.sync_copy(ones_v, acc.at[dst_v.at[j]], add=True)
      return carry

    lax.fori_loop(0, nchunk, step, 0)
    plsc.subcore_barrier()
    pltpu.sync_copy(acc.at[pl.ds(base, rpt)], out_hbm.at[cid, pl.ds(base, rpt)])

  return deg_kernel(dst_r)


def _sc_edge_sum(y, src_r, dst_r, n_pad):
  nchunk = src_r.shape[1]
  rpt = n_pad // NS

  @functools.partial(
      pl.kernel,
      out_type=jax.ShapeDtypeStruct((NC, n_pad, 64), F32),
      mesh=_sc_mesh(),
      compiler_params=pltpu.CompilerParams(use_tc_tiling_on_sc=False),
      scratch_types=[
          pltpu.VMEM((nchunk, CH), jnp.int32),
          pltpu.VMEM((nchunk, CH), jnp.int32),
          pltpu.VMEM((CH, 64), F32),
          pltpu.VMEM((CH, 64), F32),
          pltpu.VMEM((16, 64), F32),
          pltpu.VMEM_SHARED((n_pad, 64), F32),
          pltpu.SemaphoreType.DMA,
          pltpu.SemaphoreType.DMA,
      ],
  )
  def scat_kernel(y_hbm, src_hbm, dst_hbm, out_hbm, src_v, dst_v, rows0,
                  rows1, zero_v, acc, sem0, sem1):
    cid = lax.axis_index("c")
    sid = lax.axis_index("s")
    wid = sid * NC + cid
    pltpu.sync_copy(src_hbm.at[wid], src_v)
    pltpu.sync_copy(dst_hbm.at[wid], dst_v)
    for i in range(16):
      for j in range(4):
        zero_v[i, pl.ds(j * 16, 16)] = jnp.zeros((16,), F32)
    base = sid * rpt
    for k in range(rpt // 16):
      pltpu.sync_copy(zero_v, acc.at[pl.ds(base + k * 16, 16)])
    plsc.subcore_barrier()

    pltpu.async_copy(y_hbm.at[src_v.at[0]], rows0, sem0)

    def pair(i, carry):
      j = 2 * i
      pltpu.async_copy(y_hbm.at[src_v.at[j + 1]], rows1, sem1)
      pltpu.make_async_copy(y_hbm.at[pl.ds(0, CH)], rows0, sem0).wait()
      pltpu.sync_copy(rows0, acc.at[dst_v.at[j]], add=True)
      pltpu.async_copy(y_hbm.at[src_v.at[j + 2]], rows0, sem0)
      pltpu.make_async_copy(y_hbm.at[pl.ds(0, CH)], rows1, sem1).wait()
      pltpu.sync_copy(rows1, acc.at[dst_v.at[j + 1]], add=True)
      return carry

    lax.fori_loop(0, (nchunk - 1) // 2, pair, 0)
    pltpu.make_async_copy(y_hbm.at[pl.ds(0, CH)], rows0, sem0).wait()
    pltpu.sync_copy(rows0, acc.at[dst_v.at[nchunk - 1]], add=True)
    plsc.subcore_barrier()
    pltpu.sync_copy(acc.at[pl.ds(base, rpt)], out_hbm.at[cid, pl.ds(base, rpt)])

  return scat_kernel(y, src_r, dst_r)


def _stage_a(h, wb, wc, bc, wr, br, deg0, deg1, dinv_in, n_pad, first):
  grid = n_pad // BLK
  hid = h.shape[1]

  def body(*refs):
    if first:
      (h_ref, wb_ref, wc_ref, bc_ref, wr_ref, br_ref, d0_ref, d1_ref,
       y_ref, comb_ref, res_ref, dinv_ref) = refs
      deg = d0_ref[:, 0:1] + d1_ref[:, 0:1] + 1.0
      dinv = lax.rsqrt(deg)
      dinv_ref[...] = jnp.broadcast_to(dinv, (BLK, 16))
    else:
      (h_ref, wb_ref, wc_ref, bc_ref, wr_ref, br_ref, dv_ref,
       y_ref, comb_ref, res_ref) = refs
      dinv = dv_ref[:, 0:1]
    hb = h_ref[...]
    y_ref[...] = jnp.dot(hb, wb_ref[...], preferred_element_type=F32) * dinv
    comb_ref[...] = jnp.dot(hb, wc_ref[...],
                            preferred_element_type=F32) + bc_ref[...]
    res_ref[...] = jnp.dot(hb, wr_ref[...],
                           preferred_element_type=F32) + br_ref[...]

  in_specs = [
      pl.BlockSpec((BLK, hid), lambda i: (i, 0)),
      pl.BlockSpec((hid, 64), lambda i: (0, 0)),
      pl.BlockSpec((hid, 32), lambda i: (0, 0)),
      pl.BlockSpec((1, 32), lambda i: (0, 0)),
      pl.BlockSpec((hid, 128), lambda i: (0, 0)),
      pl.BlockSpec((1, 128), lambda i: (0, 0)),
  ]
  out_shape = [
      jax.ShapeDtypeStruct((n_pad, 64), F32),
      jax.ShapeDtypeStruct((n_pad, 32), F32),
      jax.ShapeDtypeStruct((n_pad, 128), F32),
  ]
  out_specs = [
      pl.BlockSpec((BLK, 64), lambda i: (i, 0)),
      pl.BlockSpec((BLK, 32), lambda i: (i, 0)),
      pl.BlockSpec((BLK, 128), lambda i: (i, 0)),
  ]
  if first:
    in_specs += [pl.BlockSpec((BLK, 16), lambda i: (i, 0)),
                 pl.BlockSpec((BLK, 16), lambda i: (i, 0))]
    out_shape.append(jax.ShapeDtypeStruct((n_pad, 16), F32))
    out_specs.append(pl.BlockSpec((BLK, 16), lambda i: (i, 0)))
    args = (h, wb, wc, bc, wr, br, deg0, deg1)
  else:
    in_specs.append(pl.BlockSpec((BLK, 16), lambda i: (i, 0)))
    args = (h, wb, wc, bc, wr, br, dinv_in)

  return pl.pallas_call(
      body, grid=(grid,), in_specs=in_specs, out_specs=out_specs,
      out_shape=out_shape)(*args)


def _combine_body(y_ref, s0_ref, s1_ref, comb_ref, res_ref, dv_ref, bias_ref,
                  g_ref, be_ref):
  dinv = dv_ref[:, 0:1]
  agg = (y_ref[...] + s0_ref[...] + s1_ref[...]) * dinv
  comb = comb_ref[...]
  conv = jnp.zeros((BLK, 128), F32)
  ri_e = lax.broadcasted_iota(jnp.int32, (32, 128), 0)
  ci_e = lax.broadcasted_iota(jnp.int32, (32, 128), 1)
  ri_f = lax.broadcasted_iota(jnp.int32, (64, 128), 0)
  ci_f = lax.broadcasted_iota(jnp.int32, (64, 128), 1)
  for b in range(B):
    eb = jnp.where((ri_e % B == b) & (ci_e // 16 == ri_e // B), 1.0, 0.0)
    fb = jnp.where((ri_f // 16 == b) & (ci_f % 16 == ri_f % 16), 1.0, 0.0)
    ce = jnp.dot(comb, eb, preferred_element_type=F32)
    ae = jnp.dot(agg, fb, preferred_element_type=F32)
    conv = conv + ce * ae
  o = conv + bias_ref[...] + res_ref[...]
  mu = jnp.mean(o, axis=1, keepdims=True)
  var = jnp.mean((o - mu) ** 2, axis=1, keepdims=True)
  hn = (o - mu) / jnp.sqrt(var + 1e-5) * g_ref[...] + be_ref[...]
  return jnp.maximum(hn, 0.0)


def _stage_d_a(y, s0, s1, comb, res, dinv, bias, g, be, wb, wc, bc, wr, br,
               n_pad):
  grid = n_pad // BLK

  def body(y_ref, s0_ref, s1_ref, comb_ref, res_ref, dv_ref, bias_ref, g_ref,
           be_ref, wb_ref, wc_ref, bc_ref, wr_ref, br_ref,
           y1_ref, comb1_ref, res1_ref):
    h1 = _combine_body(y_ref, s0_ref, s1_ref, comb_ref, res_ref, dv_ref,
                       bias_ref, g_ref, be_ref)
    dinv = dv_ref[:, 0:1]
    y1_ref[...] = jnp.dot(h1, wb_ref[...], preferred_element_type=F32) * dinv
    comb1_ref[...] = jnp.dot(h1, wc_ref[...],
                             preferred_element_type=F32) + bc_ref[...]
    res1_ref[...] = jnp.dot(h1, wr_ref[...],
                            preferred_element_type=F32) + br_ref[...]

  return pl.pallas_call(
      body, grid=(grid,),
      in_specs=[
          pl.BlockSpec((BLK, 64), lambda i: (i, 0)),
          pl.BlockSpec((BLK, 64), lambda i: (i, 0)),
          pl.BlockSpec((BLK, 64), lambda i: (i, 0)),
          pl.BlockSpec((BLK, 32), lambda i: (i, 0)),
          pl.BlockSpec((BLK, 128), lambda i: (i, 0)),
          pl.BlockSpec((BLK, 16), lambda i: (i, 0)),
          pl.BlockSpec((1, 128), lambda i: (0, 0)),
          pl.BlockSpec((1, 128), lambda i: (0, 0)),
          pl.BlockSpec((1, 128), lambda i: (0, 0)),
          pl.BlockSpec((128, 64), lambda i: (0, 0)),
          pl.BlockSpec((128, 32), lambda i: (0, 0)),
          pl.BlockSpec((1, 32), lambda i: (0, 0)),
          pl.BlockSpec((128, 128), lambda i: (0, 0)),
          pl.BlockSpec((1, 128), lambda i: (0, 0)),
      ],
      out_specs=[
          pl.BlockSpec((BLK, 64), lambda i: (i, 0)),
          pl.BlockSpec((BLK, 32), lambda i: (i, 0)),
          pl.BlockSpec((BLK, 128), lambda i: (i, 0)),
      ],
      out_shape=[
          jax.ShapeDtypeStruct((n_pad, 64), F32),
          jax.ShapeDtypeStruct((n_pad, 32), F32),
          jax.ShapeDtypeStruct((n_pad, 128), F32),
      ],
  )(y, s0, s1, comb, res, dinv, bias, g, be, wb, wc, bc, wr, br)


def _stage_d_pool(y, s0, s1, comb, res, dinv, bias, g, be, batch3, wf, n_pad):
  grid = n_pad // BLK

  def body(y_ref, s0_ref, s1_ref, comb_ref, res_ref, dv_ref, bias_ref, g_ref,
           be_ref, batch_ref, wf_ref, out_ref, sums, cnts):
    i = pl.program_id(0)

    @pl.when(i == 0)
    def _init():
      sums[...] = jnp.zeros((NUM_GRAPHS, 128), F32)
      cnts[...] = jnp.zeros((NUM_GRAPHS, 128), F32)

    h2 = _combine_body(y_ref, s0_ref, s1_ref, comb_ref, res_ref, dv_ref,
                       bias_ref, g_ref, be_ref)
    brow = batch_ref[0]
    gi = lax.broadcasted_iota(jnp.int32, (NUM_GRAPHS, BLK), 0)
    oh = jnp.where(gi == brow, 1.0, 0.0)
    sums[...] += jnp.dot(oh, h2, preferred_element_type=F32)
    cnts[...] += jnp.broadcast_to(
        jnp.sum(oh, axis=1, keepdims=True), (NUM_GRAPHS, 128))

    @pl.when(i == grid - 1)
    def _fin():
      pooled = sums[...] / jnp.maximum(cnts[...], 1.0)
      out_ref[...] = jnp.dot(pooled, wf_ref[...], preferred_element_type=F32)

  return pl.pallas_call(
      body, grid=(grid,),
      in_specs=[
          pl.BlockSpec((BLK, 64), lambda i: (i, 0)),
          pl.BlockSpec((BLK, 64), lambda i: (i, 0)),
          pl.BlockSpec((BLK, 64), lambda i: (i, 0)),
          pl.BlockSpec((BLK, 32), lambda i: (i, 0)),
          pl.BlockSpec((BLK, 128), lambda i: (i, 0)),
          pl.BlockSpec((BLK, 16), lambda i: (i, 0)),
          pl.BlockSpec((1, 128), lambda i: (0, 0)),
          pl.BlockSpec((1, 128), lambda i: (0, 0)),
          pl.BlockSpec((1, 128), lambda i: (0, 0)),
          pl.BlockSpec((1, 1, BLK), lambda i: (i, 0, 0)),
          pl.BlockSpec((128, 8), lambda i: (0, 0)),
      ],
      out_specs=pl.BlockSpec((NUM_GRAPHS, 8), lambda i: (0, 0)),
      out_shape=jax.ShapeDtypeStruct((NUM_GRAPHS, 8), F32),
      compiler_params=pltpu.CompilerParams(
          dimension_semantics=("arbitrary",)),
      scratch_shapes=[pltpu.VMEM((NUM_GRAPHS, 128), F32),
                      pltpu.VMEM((NUM_GRAPHS, 128), F32)],
  )(y, s0, s1, comb, res, dinv, bias, g, be, batch3, wf)


def kernel(x, edge_index, batch, Wb0, Wc0, bc0, bias0, Wr0, br0, g0, be0,
           Wb1, Wc1, bc1, bias1, Wr1, br1, g1, be1, Wf):
  n = x.shape[0]
  e = edge_index.shape[1]
  n_pad = ((n + BLK - 1) // BLK) * BLK
  nchunk = -(-e // (NW * CH))
  nchunk += 1 - nchunk % 2
  e_pad = NW * nchunk * CH

  xp = jnp.pad(x, ((0, n_pad - n), (0, 0)))
  batch_p = jnp.concatenate(
      [batch, jnp.full((n_pad - n,), NUM_GRAPHS, jnp.int32)])
  batch3 = batch_p.reshape(n_pad // BLK, 1, BLK)
  pad_e = n + jnp.arange(e_pad - e, dtype=jnp.int32) % (n_pad - n)
  src_r = jnp.concatenate([edge_index[0], pad_e]).reshape(NW, nchunk, CH)
  dst_r = jnp.concatenate([edge_index[1], pad_e]).reshape(NW, nchunk, CH)

  bc0r, bc1r = bc0.reshape(1, -1), bc1.reshape(1, -1)
  row = lambda v: v.reshape(1, -1)

  deg_p = _sc_degree(dst_r, n_pad)

  y0, comb0, res0, dinv = _stage_a(
      xp, Wb0, Wc0, bc0r, Wr0, row(br0), deg_p[0], deg_p[1], None, n_pad,
      first=True)
  s_p0 = _sc_edge_sum(y0, src_r, dst_r, n_pad)
  y1, comb1, res1 = _stage_d_a(
      y0, s_p0[0], s_p0[1], comb0, res0, dinv, row(bias0), row(g0), row(be0),
      Wb1, Wc1, bc1r, Wr1, row(br1), n_pad)
  s_p1 = _sc_edge_sum(y1, src_r, dst_r, n_pad)
  out = _stage_d_pool(y1, s_p1[0], s_p1[1], comb1, res1, dinv, row(bias1),
                      row(g1), row(be1), batch3, Wf, n_pad)
  return out
